# Initial kernel scaffold; baseline (speedup 1.0000x reference)
import jax
import jax.numpy as jnp
from jax.experimental import pallas as pl


def kernel(x, edge_index, batch, nfc_W, nfc_b, gc1_Wl, gc1_bl, gc1_Wr, gc1_br, gc1_att, gc1_bias, gc2_Wl, gc2_bl, gc2_Wr, gc2_br, gc2_att, gc2_bias, fc1_W, fc1_b, fc2_W, fc2_b):
    raise NotImplementedError("write your pallas kernel here")


# speedup vs baseline: 9.1360x; 9.1360x over previous
import functools

import jax
import jax.numpy as jnp
from jax import lax
from jax.experimental import pallas as pl
from jax.experimental.pallas import tpu as pltpu
from jax.experimental.pallas import tpu_sc as plsc

N = 10000
E = 320000
G = 16
D = 128
D_FC1 = 32
D_OUT = 10
NP = 10240
NC = 2
NS = 16
NW = NC * NS
EPW = E // NW
K = 80
CPW = EPW // K
TB = 512

_mesh = plsc.VectorSubcoreMesh(core_axis_name="c", subcore_axis_name="s")
_sc_params = pltpu.CompilerParams(needs_layout_passes=False, use_tc_tiling_on_sc=False)


def _leaky(v, slope):
    return jnp.maximum(v, v * slope)


@functools.partial(
    pl.kernel,
    out_type=[
        jax.ShapeDtypeStruct((NW, CPW, K), jnp.float32),
        jax.ShapeDtypeStruct((NW * 16,), jnp.float32),
    ],
    mesh=_mesh,
    compiler_params=_sc_params,
    scratch_types=[
        pltpu.VMEM((CPW, K), jnp.int32),
        pltpu.VMEM((CPW, K), jnp.int32),
        pltpu.VMEM((D,), jnp.float32),
        pltpu.VMEM((K, D), jnp.float32),
        pltpu.VMEM((K, D), jnp.float32),
        pltpu.VMEM((CPW, K), jnp.float32),
        pltpu.VMEM((16,), jnp.float32),
        pltpu.SemaphoreType.DMA,
    ],
)
def _sc_logits(xl_hbm, xr_hbm, src_hbm, dst_hbm, att_hbm,
               logits_hbm, tmax_hbm,
               src_v, dst_v, att_v, xl_v, xr_v, log_v, red_v, sem):
    wid = lax.axis_index("s") * NC + lax.axis_index("c")
    cp1 = pltpu.async_copy(src_hbm.at[wid], src_v, sem)
    cp2 = pltpu.async_copy(dst_hbm.at[wid], dst_v, sem)
    cp3 = pltpu.async_copy(att_hbm, att_v, sem)
    cp1.wait(); cp2.wait(); cp3.wait()

    lane = lax.iota(jnp.int32, 16)

    @pl.loop(0, CPW, init_carry=jnp.full((16,), -1e30, jnp.float32))
    def chunk(c, rmax):
        g1 = pltpu.async_copy(xl_hbm.at[src_v.at[c]], xl_v, sem)
        g2 = pltpu.async_copy(xr_hbm.at[dst_v.at[c]], xr_v, sem)
        g1.wait(); g2.wait()

        @pl.loop(0, K // 16)
        def egrp(e16):
            lv = jnp.zeros((16,), jnp.float32)
            for l in range(16):
                e = e16 * 16 + l
                acc = jnp.zeros((16,), jnp.float32)
                for j in range(D // 16):
                    s = xl_v[e, pl.ds(j * 16, 16)] + xr_v[e, pl.ds(j * 16, 16)]
                    acc = acc + _leaky(s, 0.2) * att_v[pl.ds(j * 16, 16)]
                lv = jnp.where(lane == l, plsc.cumsum(acc)[15], lv)
            log_v[c, pl.ds(e16 * 16, 16)] = lv

        for q in range(K // 16):
            rmax = jnp.maximum(rmax, log_v[c, pl.ds(q * 16, 16)])
        return rmax

    red_v[...] = chunk
    pltpu.sync_copy(red_v, tmax_hbm.at[pl.ds(wid * 16, 16)])
    pltpu.sync_copy(log_v, logits_hbm.at[wid])


@functools.partial(
    pl.kernel,
    out_type=[
        jax.ShapeDtypeStruct((NW, CPW, K), jnp.float32),
        jax.ShapeDtypeStruct((NC * NP,), jnp.float32),
    ],
    mesh=_mesh,
    compiler_params=_sc_params,
    scratch_types=[
        pltpu.VMEM((CPW, K), jnp.float32),
        pltpu.VMEM((CPW, K), jnp.int32),
        pltpu.VMEM((NW * 16,), jnp.float32),
        pltpu.VMEM((640,), jnp.float32),
        pltpu.VMEM_SHARED((NP,), jnp.float32),
        pltpu.SemaphoreType.DMA,
    ],
)
def _sc_denom(logits_hbm, dst_hbm, tmax_hbm,
              ex_hbm, denomp_hbm,
              log_v, dst_v, tmax_v, zero_v, spden, sem):
    cid = lax.axis_index("c")
    sid = lax.axis_index("s")
    wid = sid * NC + cid
    cp1 = pltpu.async_copy(logits_hbm.at[wid], log_v, sem)
    cp2 = pltpu.async_copy(dst_hbm.at[wid], dst_v, sem)
    cp3 = pltpu.async_copy(tmax_hbm, tmax_v, sem)
    cp1.wait(); cp2.wait(); cp3.wait()

    m = tmax_v[pl.ds(0, 16)]
    for i in range(1, NW):
        m = jnp.maximum(m, tmax_v[pl.ds(i * 16, 16)])
    gmax = m[0]
    for l in range(1, 16):
        gmax = jnp.maximum(gmax, m[l])

    @pl.loop(0, 640 // 16)
    def zer(i):
        zero_v[pl.ds(i * 16, 16)] = jnp.zeros((16,), jnp.float32)

    pltpu.sync_copy(zero_v, spden.at[pl.ds(sid * 640, 640)])
    plsc.subcore_barrier()

    @pl.loop(0, CPW)
    def chunk(c):
        for q in range(K // 16):
            lv = log_v[c, pl.ds(q * 16, 16)]
            log_v[c, pl.ds(q * 16, 16)] = jnp.exp(lv - gmax)
        pltpu.sync_copy(log_v.at[c], spden.at[dst_v.at[c]], add=True)

    plsc.subcore_barrier()
    pltpu.sync_copy(log_v, ex_hbm.at[wid])
    pltpu.sync_copy(spden.at[pl.ds(sid * 640, 640)],
                    denomp_hbm.at[pl.ds(cid * NP + sid * 640, 640)])


CPT = NW * CPW // NS
DH = D // 2
NPS = 10048
NCH8 = NPS // 8


@functools.partial(
    pl.kernel,
    out_type=jax.ShapeDtypeStruct((NC, NPS, DH), jnp.float32),
    mesh=_mesh,
    compiler_params=_sc_params,
    scratch_types=[
        pltpu.VMEM((CPT, K), jnp.int32),
        pltpu.VMEM((CPT, K), jnp.int32),
        pltpu.VMEM((CPT, K), jnp.float32),
        pltpu.VMEM((NP,), jnp.float32),
        pltpu.VMEM((NP,), jnp.float32),
        pltpu.VMEM((K, DH), jnp.float32),
        pltpu.VMEM((K, DH), jnp.float32),
        pltpu.VMEM((K,), jnp.float32),
        pltpu.VMEM_SHARED((NPS, DH), jnp.float32),
        pltpu.SemaphoreType.DMA,
    ],
)
def _sc_aggregate(xlh_hbm, src_hbm, dst_hbm, ex_hbm, denomp_hbm,
                  outp_hbm,
                  src_v, dst_v, ex_v, den_v, den2_v, xlr_v, stage_v, al_v,
                  spout, sem):
    cid = lax.axis_index("c")
    sid = lax.axis_index("s")
    cp1 = pltpu.async_copy(src_hbm.at[sid], src_v, sem)
    cp2 = pltpu.async_copy(dst_hbm.at[sid], dst_v, sem)
    cp3 = pltpu.async_copy(ex_hbm.at[sid], ex_v, sem)
    cp4 = pltpu.async_copy(denomp_hbm.at[pl.ds(0, NP)], den_v, sem)
    cp5 = pltpu.async_copy(denomp_hbm.at[pl.ds(NP, NP)], den2_v, sem)
    cp1.wait(); cp2.wait(); cp3.wait(); cp4.wait(); cp5.wait()

    @pl.loop(0, NP // 16)
    def inv(i):
        v = den_v[pl.ds(i * 16, 16)] + den2_v[pl.ds(i * 16, 16)]
        den_v[pl.ds(i * 16, 16)] = jnp.float32(1.0) / v

    @pl.loop(0, CPT)
    def fixsrc(c):
        for q in range(K // 16):
            v = src_v[c, pl.ds(q * 16, 16)]
            src_v[c, pl.ds(q * 16, 16)] = v * 2 + cid

    @pl.loop(0, 8)
    def zer(e):
        for j in range(DH // 16):
            stage_v[e, pl.ds(j * 16, 16)] = jnp.zeros((16,), jnp.float32)

    nk = (NCH8 - sid + NS - 1) // NS

    @pl.loop(0, nk)
    def zcp(k):
        pltpu.sync_copy(stage_v.at[pl.ds(0, 8)],
                        spout.at[pl.ds((sid + k * NS) * 8, 8)])

    plsc.subcore_barrier()

    @pl.loop(0, CPT)
    def chunk(c):
        g1 = pltpu.async_copy(xlh_hbm.at[src_v.at[c]], xlr_v, sem)
        g1.wait()
        for q in range(K // 16):
            didx = dst_v[c, pl.ds(q * 16, 16)]
            invd = plsc.load_gather(den_v, [didx])
            al_v[pl.ds(q * 16, 16)] = ex_v[c, pl.ds(q * 16, 16)] * invd

        @pl.loop(0, K // 16)
        def egrp(e16):
            av = al_v[pl.ds(e16 * 16, 16)]
            for l in range(16):
                e = e16 * 16 + l
                a = av[l]
                for j in range(DH // 16):
                    stage_v[e, pl.ds(j * 16, 16)] = xlr_v[e, pl.ds(j * 16, 16)] * a

        pltpu.sync_copy(stage_v, spout.at[dst_v.at[c]], add=True)

    plsc.subcore_barrier()

    @pl.loop(0, nk)
    def wcp(k):
        r0 = (sid + k * NS) * 8
        pltpu.sync_copy(spout.at[pl.ds(r0, 8)], stage_v.at[pl.ds(0, 8)])
        pltpu.sync_copy(stage_v.at[pl.ds(0, 8)], outp_hbm.at[cid, pl.ds(r0, 8)])


def _mm_t(a, w):
    return lax.dot_general(a, w, (((1,), (1,)), ((), ())),
                           preferred_element_type=jnp.float32)


def _tc_proj1_body(x_ref, W0_ref, b0_ref, Wl_ref, bl_ref, Wr_ref, br_ref,
                   xl_ref, xr_ref):
    h = _leaky(_mm_t(x_ref[...], W0_ref[...]) + b0_ref[...], 0.01)
    xl_ref[...] = _mm_t(h, Wl_ref[...]) + bl_ref[...]
    xr_ref[...] = _mm_t(h, Wr_ref[...]) + br_ref[...]


def _tc_proj2_body(lo_ref, hi_ref, bias_ref, Wl_ref, bl_ref, Wr_ref, br_ref,
                   xl_ref, xr_ref):
    h = _leaky(jnp.concatenate([lo_ref[...], hi_ref[...]], axis=1)
               + bias_ref[...], 0.01)
    xl_ref[...] = _mm_t(h, Wl_ref[...]) + bl_ref[...]
    xr_ref[...] = _mm_t(h, Wr_ref[...]) + br_ref[...]


def _tc_head_body(lo_ref, hi_ref, bias_ref, batch_ref,
                  fc1W_ref, fc1b_ref, fc2W_ref, fc2b_ref, out_ref):
    h3 = _leaky(jnp.concatenate([lo_ref[...], hi_ref[...]], axis=1)
                + bias_ref[...], 0.01)
    gid = lax.broadcasted_iota(jnp.int32, (G, 1), 0)
    onehot = (batch_ref[...] == gid).astype(jnp.float32)
    sums = lax.dot_general(onehot, h3, (((1,), (0,)), ((), ())),
                           preferred_element_type=jnp.float32)
    counts = jnp.sum(onehot, axis=1, keepdims=True)
    hg = sums / jnp.maximum(counts, 1.0)
    z1 = _leaky(_mm_t(hg, fc1W_ref[...]) + fc1b_ref[...], 0.01)
    out_ref[...] = _mm_t(z1, fc2W_ref[...]) + fc2b_ref[...]


_w_spec = pl.BlockSpec((D, D), lambda i: (0, 0))
_b_spec = pl.BlockSpec((1, D), lambda i: (0, 0))
_r_spec = pl.BlockSpec((TB, D), lambda i: (i, 0))

_tc_proj1 = pl.pallas_call(
    _tc_proj1_body,
    grid=(NP // TB,),
    in_specs=[_r_spec, _w_spec, _b_spec, _w_spec, _b_spec, _w_spec, _b_spec],
    out_specs=[_r_spec, _r_spec],
    out_shape=[jax.ShapeDtypeStruct((NP, D), jnp.float32)] * 2,
)

_h_spec = pl.BlockSpec((TB, DH), lambda i: (i, 0))

_tc_proj2 = pl.pallas_call(
    _tc_proj2_body,
    grid=(NP // TB,),
    in_specs=[_h_spec, _h_spec, _b_spec, _w_spec, _b_spec, _w_spec, _b_spec],
    out_specs=[_r_spec, _r_spec],
    out_shape=[jax.ShapeDtypeStruct((NP, D), jnp.float32)] * 2,
)

_tc_head = pl.pallas_call(
    _tc_head_body,
    out_shape=jax.ShapeDtypeStruct((G, D_OUT), jnp.float32),
)


def kernel(x, edge_index, batch, nfc_W, nfc_b,
           gc1_Wl, gc1_bl, gc1_Wr, gc1_br, gc1_att, gc1_bias,
           gc2_Wl, gc2_bl, gc2_Wr, gc2_br, gc2_att, gc2_bias,
           fc1_W, fc1_b, fc2_W, fc2_b):
    x_p = jnp.pad(x, ((0, NP - N), (0, 0)))
    src2d = edge_index[0].reshape(NW, CPW, K)
    dst2d = edge_index[1].reshape(NW, CPW, K)
    src3d = edge_index[0].reshape(NS, CPT, K)
    dst3d = edge_index[1].reshape(NS, CPT, K)
    batch2d = jnp.pad(batch, (0, NP - N), constant_values=G).reshape(1, NP)

    def b2(v):
        return v.reshape(1, -1)

    xl1, xr1 = _tc_proj1(x_p, nfc_W, b2(nfc_b), gc1_Wl, b2(gc1_bl),
                         gc1_Wr, b2(gc1_br))
    logits1, tmax1 = _sc_logits(xl1, xr1, src2d, dst2d, gc1_att)
    ex1, denp1 = _sc_denom(logits1, dst2d, tmax1)
    outp1 = _sc_aggregate(xl1.reshape(2 * NP, DH), src3d, dst3d,
                          ex1.reshape(NS, CPT, K), denp1)

    pad_rows = ((0, NP - NPS), (0, 0))
    xl2, xr2 = _tc_proj2(jnp.pad(outp1[0], pad_rows), jnp.pad(outp1[1], pad_rows),
                         b2(gc1_bias), gc2_Wl, b2(gc2_bl), gc2_Wr, b2(gc2_br))
    logits2, tmax2 = _sc_logits(xl2, xr2, src2d, dst2d, gc2_att)
    ex2, denp2 = _sc_denom(logits2, dst2d, tmax2)
    outp2 = _sc_aggregate(xl2.reshape(2 * NP, DH), src3d, dst3d,
                          ex2.reshape(NS, CPT, K), denp2)

    return _tc_head(jnp.pad(outp2[0], pad_rows), jnp.pad(outp2[1], pad_rows),
                    b2(gc2_bias), batch2d, fc1_W, b2(fc1_b), fc2_W, b2(fc2_b))


# speedup vs baseline: 15.1744x; 1.6609x over previous
import functools

import jax
import jax.numpy as jnp
from jax import lax
from jax.experimental import pallas as pl
from jax.experimental.pallas import tpu as pltpu
from jax.experimental.pallas import tpu_sc as plsc

N = 10000
E = 320000
G = 16
D = 128
D_FC1 = 32
D_OUT = 10
NP = 10240
NC = 2
NS = 16
NW = NC * NS
EPW = E // NW
K = 80
CPW = EPW // K
TB = 512

_mesh = plsc.VectorSubcoreMesh(core_axis_name="c", subcore_axis_name="s")
_sc_params = pltpu.CompilerParams(needs_layout_passes=False, use_tc_tiling_on_sc=False)


def _leaky(v, slope):
    return jnp.maximum(v, v * slope)


@functools.partial(
    pl.kernel,
    out_type=[
        jax.ShapeDtypeStruct((NW, CPW, K), jnp.float32),
        jax.ShapeDtypeStruct((NW * 16,), jnp.float32),
    ],
    mesh=_mesh,
    compiler_params=_sc_params,
    scratch_types=[
        pltpu.VMEM((CPW, K), jnp.int32),
        pltpu.VMEM((CPW, K), jnp.int32),
        pltpu.VMEM((D,), jnp.float32),
        pltpu.VMEM((K, D), jnp.float32),
        pltpu.VMEM((K, D), jnp.float32),
        pltpu.VMEM((K, D), jnp.float32),
        pltpu.VMEM((K, D), jnp.float32),
        pltpu.VMEM((CPW, K), jnp.float32),
        pltpu.VMEM((16,), jnp.float32),
        pltpu.SemaphoreType.DMA,
    ],
)
def _sc_logits(xl_hbm, xr_hbm, src_hbm, dst_hbm, att_hbm,
               logits_hbm, tmax_hbm,
               src_v, dst_v, att_v, xl0_v, xl1_v, xr0_v, xr1_v,
               log_v, red_v, sem):
    wid = lax.axis_index("s") * NC + lax.axis_index("c")
    cp1 = pltpu.async_copy(src_hbm.at[wid], src_v, sem)
    cp2 = pltpu.async_copy(dst_hbm.at[wid], dst_v, sem)
    cp3 = pltpu.async_copy(att_hbm, att_v, sem)
    cp1.wait(); cp2.wait(); cp3.wait()

    lane = lax.iota(jnp.int32, 16)

    def _compute(c, xl_v, xr_v, rmax):
        @pl.loop(0, K // 16)
        def egrp(e16):
            lv = jnp.zeros((16,), jnp.float32)
            for l in range(16):
                e = e16 * 16 + l
                acc = jnp.zeros((16,), jnp.float32)
                for j in range(D // 16):
                    s = xl_v[e, pl.ds(j * 16, 16)] + xr_v[e, pl.ds(j * 16, 16)]
                    acc = acc + _leaky(s, 0.2) * att_v[pl.ds(j * 16, 16)]
                lv = jnp.where(lane == l, plsc.cumsum(acc)[15], lv)
            log_v[c, pl.ds(e16 * 16, 16)] = lv

        for q in range(K // 16):
            rmax = jnp.maximum(rmax, log_v[c, pl.ds(q * 16, 16)])
        return rmax

    def _gather(c, xl_v, xr_v):
        pltpu.async_copy(xl_hbm.at[src_v.at[c]], xl_v, sem)
        pltpu.async_copy(xr_hbm.at[dst_v.at[c]], xr_v, sem)

    def _drain(c, xl_v, xr_v):
        pltpu.make_async_copy(xl_hbm.at[src_v.at[c]], xl_v, sem).wait()
        pltpu.make_async_copy(xr_hbm.at[dst_v.at[c]], xr_v, sem).wait()

    _gather(0, xl0_v, xr0_v)

    @pl.loop(0, CPW // 2, init_carry=jnp.full((16,), -1e30, jnp.float32))
    def pair(p, rmax):
        c0 = p * 2
        _gather(c0 + 1, xl1_v, xr1_v)
        _drain(c0, xl0_v, xr0_v)
        rmax = _compute(c0, xl0_v, xr0_v, rmax)
        _gather(c0 + 2, xl0_v, xr0_v)
        _drain(c0 + 1, xl1_v, xr1_v)
        return _compute(c0 + 1, xl1_v, xr1_v, rmax)

    _drain(CPW - 1, xl0_v, xr0_v)
    red_v[...] = _compute(CPW - 1, xl0_v, xr0_v, pair)
    pltpu.sync_copy(red_v, tmax_hbm.at[pl.ds(wid * 16, 16)])
    pltpu.sync_copy(log_v, logits_hbm.at[wid])


@functools.partial(
    pl.kernel,
    out_type=[
        jax.ShapeDtypeStruct((NW, CPW, K), jnp.float32),
        jax.ShapeDtypeStruct((NC * NP,), jnp.float32),
    ],
    mesh=_mesh,
    compiler_params=_sc_params,
    scratch_types=[
        pltpu.VMEM((CPW, K), jnp.float32),
        pltpu.VMEM((CPW, K), jnp.int32),
        pltpu.VMEM((NW * 16,), jnp.float32),
        pltpu.VMEM((640,), jnp.float32),
        pltpu.VMEM_SHARED((NP,), jnp.float32),
        pltpu.SemaphoreType.DMA,
    ],
)
def _sc_denom(logits_hbm, dst_hbm, tmax_hbm,
              ex_hbm, denomp_hbm,
              log_v, dst_v, tmax_v, zero_v, spden, sem):
    cid = lax.axis_index("c")
    sid = lax.axis_index("s")
    wid = sid * NC + cid
    cp1 = pltpu.async_copy(logits_hbm.at[wid], log_v, sem)
    cp2 = pltpu.async_copy(dst_hbm.at[wid], dst_v, sem)
    cp3 = pltpu.async_copy(tmax_hbm, tmax_v, sem)
    cp1.wait(); cp2.wait(); cp3.wait()

    m = tmax_v[pl.ds(0, 16)]
    for i in range(1, NW):
        m = jnp.maximum(m, tmax_v[pl.ds(i * 16, 16)])
    gmax = m[0]
    for l in range(1, 16):
        gmax = jnp.maximum(gmax, m[l])

    @pl.loop(0, 640 // 16)
    def zer(i):
        zero_v[pl.ds(i * 16, 16)] = jnp.zeros((16,), jnp.float32)

    pltpu.sync_copy(zero_v, spden.at[pl.ds(sid * 640, 640)])
    plsc.subcore_barrier()

    @pl.loop(0, CPW)
    def chunk(c):
        for q in range(K // 16):
            lv = log_v[c, pl.ds(q * 16, 16)]
            log_v[c, pl.ds(q * 16, 16)] = jnp.exp(lv - gmax)
        pltpu.sync_copy(log_v.at[c], spden.at[dst_v.at[c]], add=True)

    plsc.subcore_barrier()
    pltpu.sync_copy(log_v, ex_hbm.at[wid])
    pltpu.sync_copy(spden.at[pl.ds(sid * 640, 640)],
                    denomp_hbm.at[pl.ds(cid * NP + sid * 640, 640)])


CPT = NW * CPW // NS
DH = D // 2
NPS = 10048
NCH8 = NPS // 8


@functools.partial(
    pl.kernel,
    out_type=jax.ShapeDtypeStruct((NC, NPS, DH), jnp.float32),
    mesh=_mesh,
    compiler_params=_sc_params,
    scratch_types=[
        pltpu.VMEM((CPT, K), jnp.int32),
        pltpu.VMEM((CPT, K), jnp.int32),
        pltpu.VMEM((CPT, K), jnp.float32),
        pltpu.VMEM((NP,), jnp.float32),
        pltpu.VMEM((1024,), jnp.float32),
        pltpu.VMEM((K, DH), jnp.float32),
        pltpu.VMEM((K, DH), jnp.float32),
        pltpu.VMEM((K, DH), jnp.float32),
        pltpu.VMEM((K,), jnp.float32),
        pltpu.VMEM_SHARED((NPS, DH), jnp.float32),
        pltpu.SemaphoreType.DMA,
    ],
)
def _sc_aggregate(xlh_hbm, src_hbm, dst_hbm, ex_hbm, denomp_hbm,
                  outp_hbm,
                  src_v, dst_v, ex_v, den_v, den2_v, xlr0_v, xlr1_v,
                  stage_v, al_v, spout, sem):
    cid = lax.axis_index("c")
    sid = lax.axis_index("s")
    cp1 = pltpu.async_copy(src_hbm.at[sid], src_v, sem)
    cp2 = pltpu.async_copy(dst_hbm.at[sid], dst_v, sem)
    cp3 = pltpu.async_copy(ex_hbm.at[sid], ex_v, sem)
    cp4 = pltpu.async_copy(denomp_hbm.at[pl.ds(0, NP)], den_v, sem)
    cp1.wait(); cp2.wait(); cp3.wait(); cp4.wait()

    @pl.loop(0, NP // 1024)
    def inv(i):
        pltpu.sync_copy(denomp_hbm.at[pl.ds(NP + i * 1024, 1024)], den2_v)

        @pl.loop(0, 1024 // 16)
        def invj(j):
            o = i * 1024 + j * 16
            v = den_v[pl.ds(o, 16)] + den2_v[pl.ds(j * 16, 16)]
            den_v[pl.ds(o, 16)] = jnp.float32(1.0) / v

    @pl.loop(0, CPT)
    def fixsrc(c):
        for q in range(K // 16):
            v = src_v[c, pl.ds(q * 16, 16)]
            src_v[c, pl.ds(q * 16, 16)] = v * 2 + cid

    @pl.loop(0, 8)
    def zer(e):
        for j in range(DH // 16):
            stage_v[e, pl.ds(j * 16, 16)] = jnp.zeros((16,), jnp.float32)

    nk = (NCH8 - sid + NS - 1) // NS

    @pl.loop(0, nk)
    def zcp(k):
        pltpu.sync_copy(stage_v.at[pl.ds(0, 8)],
                        spout.at[pl.ds((sid + k * NS) * 8, 8)])

    pltpu.async_copy(xlh_hbm.at[src_v.at[0]], xlr0_v, sem)

    plsc.subcore_barrier()

    def _do_chunk(c, xlr_v):
        for q in range(K // 16):
            didx = dst_v[c, pl.ds(q * 16, 16)]
            invd = plsc.load_gather(den_v, [didx])
            al_v[pl.ds(q * 16, 16)] = ex_v[c, pl.ds(q * 16, 16)] * invd

        @pl.loop(0, K // 16)
        def egrp(e16):
            av = al_v[pl.ds(e16 * 16, 16)]
            for l in range(16):
                e = e16 * 16 + l
                a = av[l]
                for j in range(DH // 16):
                    stage_v[e, pl.ds(j * 16, 16)] = xlr_v[e, pl.ds(j * 16, 16)] * a

        pltpu.sync_copy(stage_v, spout.at[dst_v.at[c]], add=True)

    @pl.loop(0, CPT // 2)
    def pair(p):
        c0 = p * 2
        pltpu.async_copy(xlh_hbm.at[src_v.at[c0 + 1]], xlr1_v, sem)
        pltpu.make_async_copy(xlh_hbm.at[src_v.at[c0]], xlr0_v, sem).wait()
        _do_chunk(c0, xlr0_v)

        @pl.when(c0 + 2 < CPT)
        def pre():
            pltpu.async_copy(xlh_hbm.at[src_v.at[c0 + 2]], xlr0_v, sem)

        pltpu.make_async_copy(xlh_hbm.at[src_v.at[c0 + 1]], xlr1_v, sem).wait()
        _do_chunk(c0 + 1, xlr1_v)

    plsc.subcore_barrier()

    @pl.loop(0, nk)
    def wcp(k):
        r0 = (sid + k * NS) * 8
        pltpu.sync_copy(spout.at[pl.ds(r0, 8)], stage_v.at[pl.ds(0, 8)])
        pltpu.sync_copy(stage_v.at[pl.ds(0, 8)], outp_hbm.at[cid, pl.ds(r0, 8)])


def _mm_t(a, w):
    return lax.dot_general(a, w, (((1,), (1,)), ((), ())),
                           preferred_element_type=jnp.float32)


def _tc_proj1_body(x_ref, W0_ref, b0_ref, Wl_ref, bl_ref, Wr_ref, br_ref,
                   xl_ref, xr_ref):
    h = _leaky(_mm_t(x_ref[...], W0_ref[...]) + b0_ref[...], 0.01)
    xl_ref[...] = _mm_t(h, Wl_ref[...]) + bl_ref[...]
    xr_ref[...] = _mm_t(h, Wr_ref[...]) + br_ref[...]


def _tc_proj2_body(lo_ref, hi_ref, bias_ref, Wl_ref, bl_ref, Wr_ref, br_ref,
                   xl_ref, xr_ref):
    h = _leaky(jnp.concatenate([lo_ref[...], hi_ref[...]], axis=1)
               + bias_ref[...], 0.01)
    xl_ref[...] = _mm_t(h, Wl_ref[...]) + bl_ref[...]
    xr_ref[...] = _mm_t(h, Wr_ref[...]) + br_ref[...]


def _tc_head_body(lo_ref, hi_ref, bias_ref, batch_ref,
                  fc1W_ref, fc1b_ref, fc2W_ref, fc2b_ref, out_ref):
    h3 = _leaky(jnp.concatenate([lo_ref[...], hi_ref[...]], axis=1)
                + bias_ref[...], 0.01)
    gid = lax.broadcasted_iota(jnp.int32, (G, 1), 0)
    onehot = (batch_ref[...] == gid).astype(jnp.float32)
    sums = lax.dot_general(onehot, h3, (((1,), (0,)), ((), ())),
                           preferred_element_type=jnp.float32)
    counts = jnp.sum(onehot, axis=1, keepdims=True)
    hg = sums / jnp.maximum(counts, 1.0)
    z1 = _leaky(_mm_t(hg, fc1W_ref[...]) + fc1b_ref[...], 0.01)
    out_ref[...] = _mm_t(z1, fc2W_ref[...]) + fc2b_ref[...]


_w_spec = pl.BlockSpec((D, D), lambda i: (0, 0))
_b_spec = pl.BlockSpec((1, D), lambda i: (0, 0))
_r_spec = pl.BlockSpec((TB, D), lambda i: (i, 0))

_tc_proj1 = pl.pallas_call(
    _tc_proj1_body,
    grid=(NP // TB,),
    in_specs=[_r_spec, _w_spec, _b_spec, _w_spec, _b_spec, _w_spec, _b_spec],
    out_specs=[_r_spec, _r_spec],
    out_shape=[jax.ShapeDtypeStruct((NP, D), jnp.float32)] * 2,
)

_h_spec = pl.BlockSpec((TB, DH), lambda i: (i, 0))

_tc_proj2 = pl.pallas_call(
    _tc_proj2_body,
    grid=(NP // TB,),
    in_specs=[_h_spec, _h_spec, _b_spec, _w_spec, _b_spec, _w_spec, _b_spec],
    out_specs=[_r_spec, _r_spec],
    out_shape=[jax.ShapeDtypeStruct((NP, D), jnp.float32)] * 2,
)

_tc_head = pl.pallas_call(
    _tc_head_body,
    out_shape=jax.ShapeDtypeStruct((G, D_OUT), jnp.float32),
)


def kernel(x, edge_index, batch, nfc_W, nfc_b,
           gc1_Wl, gc1_bl, gc1_Wr, gc1_br, gc1_att, gc1_bias,
           gc2_Wl, gc2_bl, gc2_Wr, gc2_br, gc2_att, gc2_bias,
           fc1_W, fc1_b, fc2_W, fc2_b):
    x_p = jnp.pad(x, ((0, NP - N), (0, 0)))
    src2d = edge_index[0].reshape(NW, CPW, K)
    dst2d = edge_index[1].reshape(NW, CPW, K)
    src3d = edge_index[0].reshape(NS, CPT, K)
    dst3d = edge_index[1].reshape(NS, CPT, K)
    batch2d = jnp.pad(batch, (0, NP - N), constant_values=G).reshape(1, NP)

    def b2(v):
        return v.reshape(1, -1)

    xl1, xr1 = _tc_proj1(x_p, nfc_W, b2(nfc_b), gc1_Wl, b2(gc1_bl),
                         gc1_Wr, b2(gc1_br))
    logits1, tmax1 = _sc_logits(xl1, xr1, src2d, dst2d, gc1_att)
    ex1, denp1 = _sc_denom(logits1, dst2d, tmax1)
    outp1 = _sc_aggregate(xl1.reshape(2 * NP, DH), src3d, dst3d,
                          ex1.reshape(NS, CPT, K), denp1)

    pad_rows = ((0, NP - NPS), (0, 0))
    xl2, xr2 = _tc_proj2(jnp.pad(outp1[0], pad_rows), jnp.pad(outp1[1], pad_rows),
                         b2(gc1_bias), gc2_Wl, b2(gc2_bl), gc2_Wr, b2(gc2_br))
    logits2, tmax2 = _sc_logits(xl2, xr2, src2d, dst2d, gc2_att)
    ex2, denp2 = _sc_denom(logits2, dst2d, tmax2)
    outp2 = _sc_aggregate(xl2.reshape(2 * NP, DH), src3d, dst3d,
                          ex2.reshape(NS, CPT, K), denp2)

    return _tc_head(jnp.pad(outp2[0], pad_rows), jnp.pad(outp2[1], pad_rows),
                    b2(gc2_bias), batch2d, fc1_W, b2(fc1_b), fc2_W, b2(fc2_b))


# speedup vs baseline: 17.4571x; 1.1504x over previous
import functools

import jax
import jax.numpy as jnp
from jax import lax
from jax.experimental import pallas as pl
from jax.experimental.pallas import tpu as pltpu
from jax.experimental.pallas import tpu_sc as plsc

N = 10000
E = 320000
G = 16
D = 128
D_FC1 = 32
D_OUT = 10
NP = 10240
NC = 2
NS = 16
NW = NC * NS
EPW = E // NW
K = 80
CPW = EPW // K
TB = 512

_mesh = plsc.VectorSubcoreMesh(core_axis_name="c", subcore_axis_name="s")
_sc_params = pltpu.CompilerParams(needs_layout_passes=False, use_tc_tiling_on_sc=False)


def _leaky(v, slope):
    return jnp.maximum(v, v * slope)


@functools.partial(
    pl.kernel,
    out_type=[
        jax.ShapeDtypeStruct((NW, CPW, K), jnp.float32),
        jax.ShapeDtypeStruct((NW * 16,), jnp.float32),
    ],
    mesh=_mesh,
    compiler_params=_sc_params,
    scratch_types=[
        pltpu.VMEM((CPW, K), jnp.int32),
        pltpu.VMEM((CPW, K), jnp.int32),
        pltpu.VMEM((D,), jnp.float32),
        pltpu.VMEM((K, D), jnp.float32),
        pltpu.VMEM((K, D), jnp.float32),
        pltpu.VMEM((K, D), jnp.float32),
        pltpu.VMEM((K, D), jnp.float32),
        pltpu.VMEM((CPW, K), jnp.float32),
        pltpu.VMEM((16,), jnp.float32),
        pltpu.SemaphoreType.DMA,
    ],
)
def _sc_logits(xl_hbm, xr_hbm, src_hbm, dst_hbm, att_hbm,
               logits_hbm, tmax_hbm,
               src_v, dst_v, att_v, xl0_v, xl1_v, xr0_v, xr1_v,
               log_v, red_v, sem):
    wid = lax.axis_index("s") * NC + lax.axis_index("c")
    cp1 = pltpu.async_copy(src_hbm.at[wid], src_v, sem)
    cp2 = pltpu.async_copy(dst_hbm.at[wid], dst_v, sem)
    cp3 = pltpu.async_copy(att_hbm, att_v, sem)
    cp1.wait(); cp2.wait(); cp3.wait()

    lane = lax.iota(jnp.int32, 16)

    def _compute(c, xl_v, xr_v, rmax):
        @pl.loop(0, K // 16)
        def egrp(e16):
            lv = jnp.zeros((16,), jnp.float32)
            for l in range(16):
                e = e16 * 16 + l
                acc = jnp.zeros((16,), jnp.float32)
                for j in range(D // 16):
                    s = xl_v[e, pl.ds(j * 16, 16)] + xr_v[e, pl.ds(j * 16, 16)]
                    acc = acc + _leaky(s, 0.2) * att_v[pl.ds(j * 16, 16)]
                lv = jnp.where(lane == l, plsc.cumsum(acc)[15], lv)
            log_v[c, pl.ds(e16 * 16, 16)] = lv

        for q in range(K // 16):
            rmax = jnp.maximum(rmax, log_v[c, pl.ds(q * 16, 16)])
        return rmax

    def _gather(c, xl_v, xr_v):
        pltpu.async_copy(xl_hbm.at[src_v.at[c]], xl_v, sem)
        pltpu.async_copy(xr_hbm.at[dst_v.at[c]], xr_v, sem)

    def _drain(c, xl_v, xr_v):
        pltpu.make_async_copy(xl_hbm.at[src_v.at[c]], xl_v, sem).wait()
        pltpu.make_async_copy(xr_hbm.at[dst_v.at[c]], xr_v, sem).wait()

    _gather(0, xl0_v, xr0_v)

    @pl.loop(0, CPW // 2, init_carry=jnp.full((16,), -1e30, jnp.float32))
    def pair(p, rmax):
        c0 = p * 2
        _gather(c0 + 1, xl1_v, xr1_v)
        _drain(c0, xl0_v, xr0_v)
        rmax = _compute(c0, xl0_v, xr0_v, rmax)
        _gather(c0 + 2, xl0_v, xr0_v)
        _drain(c0 + 1, xl1_v, xr1_v)
        return _compute(c0 + 1, xl1_v, xr1_v, rmax)

    _drain(CPW - 1, xl0_v, xr0_v)
    red_v[...] = _compute(CPW - 1, xl0_v, xr0_v, pair)
    pltpu.sync_copy(red_v, tmax_hbm.at[pl.ds(wid * 16, 16)])
    pltpu.sync_copy(log_v, logits_hbm.at[wid])


@functools.partial(
    pl.kernel,
    out_type=[
        jax.ShapeDtypeStruct((NW, CPW, K), jnp.float32),
        jax.ShapeDtypeStruct((NC * NP,), jnp.float32),
    ],
    mesh=_mesh,
    compiler_params=_sc_params,
    scratch_types=[
        pltpu.VMEM((CPW, K), jnp.float32),
        pltpu.VMEM((CPW, K), jnp.int32),
        pltpu.VMEM((NW * 16,), jnp.float32),
        pltpu.VMEM((640,), jnp.float32),
        pltpu.VMEM_SHARED((NP,), jnp.float32),
        pltpu.SemaphoreType.DMA,
    ],
)
def _sc_denom(logits_hbm, dst_hbm, tmax_hbm,
              ex_hbm, denomp_hbm,
              log_v, dst_v, tmax_v, zero_v, spden, sem):
    cid = lax.axis_index("c")
    sid = lax.axis_index("s")
    wid = sid * NC + cid
    cp1 = pltpu.async_copy(logits_hbm.at[wid], log_v, sem)
    cp2 = pltpu.async_copy(dst_hbm.at[wid], dst_v, sem)
    cp3 = pltpu.async_copy(tmax_hbm, tmax_v, sem)
    cp1.wait(); cp2.wait(); cp3.wait()

    m = tmax_v[pl.ds(0, 16)]
    for i in range(1, NW):
        m = jnp.maximum(m, tmax_v[pl.ds(i * 16, 16)])
    gmax = m[0]
    for l in range(1, 16):
        gmax = jnp.maximum(gmax, m[l])

    @pl.loop(0, 640 // 16)
    def zer(i):
        zero_v[pl.ds(i * 16, 16)] = jnp.zeros((16,), jnp.float32)

    pltpu.sync_copy(zero_v, spden.at[pl.ds(sid * 640, 640)])
    plsc.subcore_barrier()

    @pl.loop(0, CPW)
    def chunk(c):
        for q in range(K // 16):
            lv = log_v[c, pl.ds(q * 16, 16)]
            log_v[c, pl.ds(q * 16, 16)] = jnp.exp(lv - gmax)
        pltpu.sync_copy(log_v.at[c], spden.at[dst_v.at[c]], add=True)

    plsc.subcore_barrier()
    pltpu.sync_copy(log_v, ex_hbm.at[wid])
    pltpu.sync_copy(spden.at[pl.ds(sid * 640, 640)],
                    denomp_hbm.at[pl.ds(cid * NP + sid * 640, 640)])


CPT = NW * CPW // NS
DH = D // 2
NPB = NP // NS
WB = NPB // K


@functools.partial(
    pl.kernel,
    out_type=jax.ShapeDtypeStruct((NC, NP, DH), jnp.float32),
    mesh=_mesh,
    compiler_params=_sc_params,
    scratch_types=[
        pltpu.VMEM((CPT, K), jnp.int32),
        pltpu.VMEM((CPT, K), jnp.int32),
        pltpu.VMEM((CPT, K), jnp.float32),
        pltpu.VMEM((NPB,), jnp.float32),
        pltpu.VMEM((NPB,), jnp.float32),
        pltpu.VMEM((K, DH), jnp.float32),
        pltpu.VMEM((K, DH), jnp.float32),
        pltpu.VMEM((K, DH), jnp.float32),
        pltpu.VMEM((K, DH), jnp.float32),
        pltpu.VMEM_SHARED((NP, DH), jnp.float32),
        pltpu.SemaphoreType.DMA,
        pltpu.SemaphoreType.DMA,
    ],
)
def _sc_aggregate(xlh_hbm, src_hbm, dst_hbm, ex_hbm, denomp_hbm,
                  outp_hbm,
                  src_v, dst_v, ex_v, den0_v, den1_v, xlr0_v, xlr1_v,
                  stage0_v, stage1_v, spout, sem, sem2):
    cid = lax.axis_index("c")
    sid = lax.axis_index("s")
    r0 = sid * NPB
    cp1 = pltpu.async_copy(src_hbm.at[sid], src_v, sem)
    cp2 = pltpu.async_copy(dst_hbm.at[sid], dst_v, sem)
    cp3 = pltpu.async_copy(ex_hbm.at[sid], ex_v, sem)
    cp4 = pltpu.async_copy(denomp_hbm.at[pl.ds(r0, NPB)], den0_v, sem)
    cp5 = pltpu.async_copy(denomp_hbm.at[pl.ds(NP + r0, NPB)], den1_v, sem)
    cp1.wait(); cp2.wait(); cp3.wait(); cp4.wait(); cp5.wait()

    @pl.loop(0, NPB // 16)
    def inv(i):
        v = den0_v[pl.ds(i * 16, 16)] + den1_v[pl.ds(i * 16, 16)]
        den0_v[pl.ds(i * 16, 16)] = jnp.float32(1.0) / (v + jnp.float32(1e-16))

    @pl.loop(0, CPT)
    def fixsrc(c):
        for q in range(K // 16):
            v = src_v[c, pl.ds(q * 16, 16)]
            src_v[c, pl.ds(q * 16, 16)] = v * 2 + cid

    @pl.loop(0, K)
    def zer(e):
        for j in range(DH // 16):
            stage0_v[e, pl.ds(j * 16, 16)] = jnp.zeros((16,), jnp.float32)

    for b in range(WB):
        pltpu.sync_copy(stage0_v, spout.at[pl.ds(r0 + b * K, K)])

    pltpu.async_copy(xlh_hbm.at[src_v.at[0]], xlr0_v, sem)

    plsc.subcore_barrier()

    def _drain_g(buf):
        pltpu.make_async_copy(xlh_hbm.at[src_v.at[0]], buf, sem).wait()

    def _drain_s(buf):
        pltpu.make_async_copy(xlh_hbm.at[src_v.at[0]], buf, sem2).wait()

    def _do_chunk(c, xlr_v, stage_v):
        @pl.loop(0, K // 16)
        def egrp(e16):
            av = ex_v[c, pl.ds(e16 * 16, 16)]
            for l in range(16):
                e = e16 * 16 + l
                a = av[l]
                for j in range(DH // 16):
                    stage_v[e, pl.ds(j * 16, 16)] = xlr_v[e, pl.ds(j * 16, 16)] * a

        pltpu.async_copy(stage_v, spout.at[dst_v.at[c]], sem2, add=True)

    @pl.loop(0, CPT // 2)
    def pair(p):
        c0 = p * 2
        pltpu.async_copy(xlh_hbm.at[src_v.at[c0 + 1]], xlr1_v, sem)
        _drain_g(xlr0_v)

        @pl.when(p > 0)
        def dr0():
            _drain_s(stage0_v)

        _do_chunk(c0, xlr0_v, stage0_v)

        @pl.when(c0 + 2 < CPT)
        def pre():
            pltpu.async_copy(xlh_hbm.at[src_v.at[c0 + 2]], xlr0_v, sem)

        _drain_g(xlr1_v)

        @pl.when(p > 0)
        def dr1():
            _drain_s(stage1_v)

        _do_chunk(c0 + 1, xlr1_v, stage1_v)

    _drain_s(stage0_v)
    _drain_s(stage1_v)
    plsc.subcore_barrier()

    for b in range(WB):
        stg = stage0_v if b % 2 == 0 else stage1_v
        if b >= 2:
            _drain_s(stg)
        pltpu.sync_copy(spout.at[pl.ds(r0 + b * K, K)], stg)
        for g in range(K // 16):
            iv = den0_v[pl.ds(b * K + g * 16, 16)]
            for l in range(16):
                e = g * 16 + l
                a = iv[l]
                for j in range(DH // 16):
                    stg[e, pl.ds(j * 16, 16)] = stg[e, pl.ds(j * 16, 16)] * a
        pltpu.async_copy(stg, outp_hbm.at[cid, pl.ds(r0 + b * K, K)], sem2)

    _drain_s(stage0_v)
    _drain_s(stage1_v)


def _mm_t(a, w):
    return lax.dot_general(a, w, (((1,), (1,)), ((), ())),
                           preferred_element_type=jnp.float32)


def _tc_proj1_body(x_ref, W0_ref, b0_ref, Wl_ref, bl_ref, Wr_ref, br_ref,
                   xl_ref, xr_ref):
    h = _leaky(_mm_t(x_ref[...], W0_ref[...]) + b0_ref[...], 0.01)
    xl_ref[...] = _mm_t(h, Wl_ref[...]) + bl_ref[...]
    xr_ref[...] = _mm_t(h, Wr_ref[...]) + br_ref[...]


def _tc_proj2_body(lo_ref, hi_ref, bias_ref, Wl_ref, bl_ref, Wr_ref, br_ref,
                   xl_ref, xr_ref):
    h = _leaky(jnp.concatenate([lo_ref[...], hi_ref[...]], axis=1)
               + bias_ref[...], 0.01)
    xl_ref[...] = _mm_t(h, Wl_ref[...]) + bl_ref[...]
    xr_ref[...] = _mm_t(h, Wr_ref[...]) + br_ref[...]


def _tc_head_body(lo_ref, hi_ref, bias_ref, batch_ref,
                  fc1W_ref, fc1b_ref, fc2W_ref, fc2b_ref, out_ref):
    h3 = _leaky(jnp.concatenate([lo_ref[...], hi_ref[...]], axis=1)
                + bias_ref[...], 0.01)
    gid = lax.broadcasted_iota(jnp.int32, (G, 1), 0)
    onehot = (batch_ref[...] == gid).astype(jnp.float32)
    sums = lax.dot_general(onehot, h3, (((1,), (0,)), ((), ())),
                           preferred_element_type=jnp.float32)
    counts = jnp.sum(onehot, axis=1, keepdims=True)
    hg = sums / jnp.maximum(counts, 1.0)
    z1 = _leaky(_mm_t(hg, fc1W_ref[...]) + fc1b_ref[...], 0.01)
    out_ref[...] = _mm_t(z1, fc2W_ref[...]) + fc2b_ref[...]


_w_spec = pl.BlockSpec((D, D), lambda i: (0, 0))
_b_spec = pl.BlockSpec((1, D), lambda i: (0, 0))
_r_spec = pl.BlockSpec((TB, D), lambda i: (i, 0))

_tc_proj1 = pl.pallas_call(
    _tc_proj1_body,
    grid=(NP // TB,),
    in_specs=[_r_spec, _w_spec, _b_spec, _w_spec, _b_spec, _w_spec, _b_spec],
    out_specs=[_r_spec, _r_spec],
    out_shape=[jax.ShapeDtypeStruct((NP, D), jnp.float32)] * 2,
)

_h_spec = pl.BlockSpec((TB, DH), lambda i: (i, 0))

_tc_proj2 = pl.pallas_call(
    _tc_proj2_body,
    grid=(NP // TB,),
    in_specs=[_h_spec, _h_spec, _b_spec, _w_spec, _b_spec, _w_spec, _b_spec],
    out_specs=[_r_spec, _r_spec],
    out_shape=[jax.ShapeDtypeStruct((NP, D), jnp.float32)] * 2,
)

_tc_head = pl.pallas_call(
    _tc_head_body,
    out_shape=jax.ShapeDtypeStruct((G, D_OUT), jnp.float32),
)


def kernel(x, edge_index, batch, nfc_W, nfc_b,
           gc1_Wl, gc1_bl, gc1_Wr, gc1_br, gc1_att, gc1_bias,
           gc2_Wl, gc2_bl, gc2_Wr, gc2_br, gc2_att, gc2_bias,
           fc1_W, fc1_b, fc2_W, fc2_b):
    x_p = jnp.pad(x, ((0, NP - N), (0, 0)))
    src2d = edge_index[0].reshape(NW, CPW, K)
    dst2d = edge_index[1].reshape(NW, CPW, K)
    src3d = edge_index[0].reshape(NS, CPT, K)
    dst3d = edge_index[1].reshape(NS, CPT, K)
    batch2d = jnp.pad(batch, (0, NP - N), constant_values=G).reshape(1, NP)

    def b2(v):
        return v.reshape(1, -1)

    xl1, xr1 = _tc_proj1(x_p, nfc_W, b2(nfc_b), gc1_Wl, b2(gc1_bl),
                         gc1_Wr, b2(gc1_br))
    logits1, tmax1 = _sc_logits(xl1, xr1, src2d, dst2d, gc1_att)
    ex1, denp1 = _sc_denom(logits1, dst2d, tmax1)
    outp1 = _sc_aggregate(xl1.reshape(2 * NP, DH), src3d, dst3d,
                          ex1.reshape(NS, CPT, K), denp1)

    xl2, xr2 = _tc_proj2(outp1[0], outp1[1],
                         b2(gc1_bias), gc2_Wl, b2(gc2_bl), gc2_Wr, b2(gc2_br))
    logits2, tmax2 = _sc_logits(xl2, xr2, src2d, dst2d, gc2_att)
    ex2, denp2 = _sc_denom(logits2, dst2d, tmax2)
    outp2 = _sc_aggregate(xl2.reshape(2 * NP, DH), src3d, dst3d,
                          ex2.reshape(NS, CPT, K), denp2)

    return _tc_head(outp2[0], outp2[1],
                    b2(gc2_bias), batch2d, fc1_W, b2(fc1_b), fc2_W, b2(fc2_b))


# speedup vs baseline: 17.6528x; 1.0112x over previous
import functools

import jax
import jax.numpy as jnp
from jax import lax
from jax.experimental import pallas as pl
from jax.experimental.pallas import tpu as pltpu
from jax.experimental.pallas import tpu_sc as plsc

N = 10000
E = 320000
G = 16
D = 128
D_FC1 = 32
D_OUT = 10
NP = 10240
NC = 2
NS = 16
NW = NC * NS
EPW = E // NW
K = 80
CPW = EPW // K
TB = 512

_mesh = plsc.VectorSubcoreMesh(core_axis_name="c", subcore_axis_name="s")
_sc_params = pltpu.CompilerParams(needs_layout_passes=False, use_tc_tiling_on_sc=False)


def _leaky(v, slope):
    return jnp.maximum(v, v * slope)


@functools.partial(
    pl.kernel,
    out_type=[
        jax.ShapeDtypeStruct((NW, CPW, K), jnp.float32),
        jax.ShapeDtypeStruct((NW * 16,), jnp.float32),
    ],
    mesh=_mesh,
    compiler_params=_sc_params,
    scratch_types=[
        pltpu.VMEM((CPW, K), jnp.int32),
        pltpu.VMEM((CPW, K), jnp.int32),
        pltpu.VMEM((D,), jnp.float32),
        pltpu.VMEM((K, D), jnp.float32),
        pltpu.VMEM((K, D), jnp.float32),
        pltpu.VMEM((K, D), jnp.float32),
        pltpu.VMEM((K, D), jnp.float32),
        pltpu.VMEM((CPW, K), jnp.float32),
        pltpu.VMEM((16,), jnp.float32),
        pltpu.SemaphoreType.DMA,
    ],
)
def _sc_logits(xl_hbm, xr_hbm, src_hbm, dst_hbm, att_hbm,
               logits_hbm, tmax_hbm,
               src_v, dst_v, att_v, xl0_v, xl1_v, xr0_v, xr1_v,
               log_v, red_v, sem):
    wid = lax.axis_index("s") * NC + lax.axis_index("c")
    cp1 = pltpu.async_copy(src_hbm.at[wid], src_v, sem)
    cp2 = pltpu.async_copy(dst_hbm.at[wid], dst_v, sem)
    cp3 = pltpu.async_copy(att_hbm, att_v, sem)
    cp1.wait(); cp2.wait(); cp3.wait()

    lane = lax.iota(jnp.int32, 16)

    def _compute(c, xl_v, xr_v, rmax):
        @pl.loop(0, K // 16)
        def egrp(e16):
            lv = jnp.zeros((16,), jnp.float32)
            for l in range(16):
                e = e16 * 16 + l
                acc = jnp.zeros((16,), jnp.float32)
                for j in range(D // 16):
                    s = xl_v[e, pl.ds(j * 16, 16)] + xr_v[e, pl.ds(j * 16, 16)]
                    acc = acc + _leaky(s, 0.2) * att_v[pl.ds(j * 16, 16)]
                lv = jnp.where(lane == l, plsc.cumsum(acc)[15], lv)
            log_v[c, pl.ds(e16 * 16, 16)] = lv

        for q in range(K // 16):
            rmax = jnp.maximum(rmax, log_v[c, pl.ds(q * 16, 16)])
        return rmax

    def _gather(c, xl_v, xr_v):
        pltpu.async_copy(xl_hbm.at[src_v.at[c]], xl_v, sem)
        pltpu.async_copy(xr_hbm.at[dst_v.at[c]], xr_v, sem)

    def _drain(c, xl_v, xr_v):
        pltpu.make_async_copy(xl_hbm.at[src_v.at[c]], xl_v, sem).wait()
        pltpu.make_async_copy(xr_hbm.at[dst_v.at[c]], xr_v, sem).wait()

    _gather(0, xl0_v, xr0_v)

    @pl.loop(0, CPW // 2, init_carry=jnp.full((16,), -1e30, jnp.float32))
    def pair(p, rmax):
        c0 = p * 2
        _gather(c0 + 1, xl1_v, xr1_v)
        _drain(c0, xl0_v, xr0_v)
        rmax = _compute(c0, xl0_v, xr0_v, rmax)
        _gather(c0 + 2, xl0_v, xr0_v)
        _drain(c0 + 1, xl1_v, xr1_v)
        return _compute(c0 + 1, xl1_v, xr1_v, rmax)

    _drain(CPW - 1, xl0_v, xr0_v)
    red_v[...] = _compute(CPW - 1, xl0_v, xr0_v, pair)
    pltpu.sync_copy(red_v, tmax_hbm.at[pl.ds(wid * 16, 16)])
    pltpu.sync_copy(log_v, logits_hbm.at[wid])


CPT = NW * CPW // NS
DH = D // 2
NPB = NP // NS
WB = NPB // K


@functools.partial(
    pl.kernel,
    out_type=jax.ShapeDtypeStruct((NC, NP, DH), jnp.float32),
    mesh=_mesh,
    compiler_params=_sc_params,
    scratch_types=[
        pltpu.VMEM((CPT, K), jnp.int32),
        pltpu.VMEM((CPT, K), jnp.int32),
        pltpu.VMEM((CPT, K), jnp.float32),
        pltpu.VMEM((NW * 16,), jnp.float32),
        pltpu.VMEM((NPB,), jnp.float32),
        pltpu.VMEM((K, DH), jnp.float32),
        pltpu.VMEM((K, DH), jnp.float32),
        pltpu.VMEM((K, DH), jnp.float32),
        pltpu.VMEM((K, DH), jnp.float32),
        pltpu.VMEM_SHARED((NP, DH), jnp.float32),
        pltpu.VMEM_SHARED((NP,), jnp.float32),
        pltpu.SemaphoreType.DMA,
        pltpu.SemaphoreType.DMA,
    ],
)
def _sc_aggregate(xlh_hbm, src_hbm, dst_hbm, logits_hbm, tmax_hbm,
                  outp_hbm,
                  src_v, dst_v, ex_v, tmax_v, den_v, xlr0_v, xlr1_v,
                  stage0_v, stage1_v, spout, spden, sem, sem2):
    cid = lax.axis_index("c")
    sid = lax.axis_index("s")
    r0 = sid * NPB
    cp1 = pltpu.async_copy(src_hbm.at[sid], src_v, sem)
    cp2 = pltpu.async_copy(dst_hbm.at[sid], dst_v, sem)
    cp3 = pltpu.async_copy(logits_hbm.at[sid], ex_v, sem)
    cp4 = pltpu.async_copy(tmax_hbm, tmax_v, sem)
    cp1.wait(); cp2.wait(); cp3.wait(); cp4.wait()

    m = tmax_v[pl.ds(0, 16)]
    for i in range(1, NW):
        m = jnp.maximum(m, tmax_v[pl.ds(i * 16, 16)])
    gmax = m[0]
    for l in range(1, 16):
        gmax = jnp.maximum(gmax, m[l])

    @pl.loop(0, CPT)
    def toex(c):
        for q in range(K // 16):
            lv = ex_v[c, pl.ds(q * 16, 16)]
            ex_v[c, pl.ds(q * 16, 16)] = jnp.exp(lv - gmax)

    @pl.loop(0, CPT)
    def fixsrc(c):
        for q in range(K // 16):
            v = src_v[c, pl.ds(q * 16, 16)]
            src_v[c, pl.ds(q * 16, 16)] = v * 2 + cid

    @pl.loop(0, K)
    def zer(e):
        for j in range(DH // 16):
            stage0_v[e, pl.ds(j * 16, 16)] = jnp.zeros((16,), jnp.float32)

    @pl.loop(0, NPB // 16)
    def zerd(i):
        den_v[pl.ds(i * 16, 16)] = jnp.zeros((16,), jnp.float32)

    for b in range(WB):
        pltpu.sync_copy(stage0_v, spout.at[pl.ds(r0 + b * K, K)])
    pltpu.sync_copy(den_v, spden.at[pl.ds(r0, NPB)])

    pltpu.async_copy(xlh_hbm.at[src_v.at[0]], xlr0_v, sem)

    plsc.subcore_barrier()

    def _drain_g(buf):
        pltpu.make_async_copy(xlh_hbm.at[src_v.at[0]], buf, sem).wait()

    def _drain_s(buf):
        pltpu.make_async_copy(xlh_hbm.at[src_v.at[0]], buf, sem2).wait()

    def _do_chunk(c, xlr_v, stage_v):
        @pl.loop(0, K // 16)
        def egrp(e16):
            av = ex_v[c, pl.ds(e16 * 16, 16)]
            for l in range(16):
                e = e16 * 16 + l
                a = av[l]
                for j in range(DH // 16):
                    stage_v[e, pl.ds(j * 16, 16)] = xlr_v[e, pl.ds(j * 16, 16)] * a

        pltpu.sync_copy(ex_v.at[c], spden.at[dst_v.at[c]], add=True)

        pltpu.async_copy(stage_v, spout.at[dst_v.at[c]], sem2, add=True)

    @pl.loop(0, CPT // 2)
    def pair(p):
        c0 = p * 2
        pltpu.async_copy(xlh_hbm.at[src_v.at[c0 + 1]], xlr1_v, sem)
        _drain_g(xlr0_v)

        @pl.when(p > 0)
        def dr0():
            _drain_s(stage0_v)

        _do_chunk(c0, xlr0_v, stage0_v)

        @pl.when(c0 + 2 < CPT)
        def pre():
            pltpu.async_copy(xlh_hbm.at[src_v.at[c0 + 2]], xlr0_v, sem)

        _drain_g(xlr1_v)

        @pl.when(p > 0)
        def dr1():
            _drain_s(stage1_v)

        _do_chunk(c0 + 1, xlr1_v, stage1_v)

    _drain_s(stage0_v)
    _drain_s(stage1_v)
    plsc.subcore_barrier()

    pltpu.sync_copy(spden.at[pl.ds(r0, NPB)], den_v)

    @pl.loop(0, NPB // 16)
    def inv(i):
        v = den_v[pl.ds(i * 16, 16)]
        den_v[pl.ds(i * 16, 16)] = jnp.float32(1.0) / (v + jnp.float32(1e-16))

    for b in range(WB):
        stg = stage0_v if b % 2 == 0 else stage1_v
        if b >= 2:
            _drain_s(stg)
        pltpu.sync_copy(spout.at[pl.ds(r0 + b * K, K)], stg)
        for g in range(K // 16):
            iv = den_v[pl.ds(b * K + g * 16, 16)]
            for l in range(16):
                e = g * 16 + l
                a = iv[l]
                for j in range(DH // 16):
                    stg[e, pl.ds(j * 16, 16)] = stg[e, pl.ds(j * 16, 16)] * a
        pltpu.async_copy(stg, outp_hbm.at[cid, pl.ds(r0 + b * K, K)], sem2)

    _drain_s(stage0_v)
    _drain_s(stage1_v)


def _mm_t(a, w):
    return lax.dot_general(a, w, (((1,), (1,)), ((), ())),
                           preferred_element_type=jnp.float32)


def _tc_proj1_body(x_ref, W0_ref, b0_ref, Wl_ref, bl_ref, Wr_ref, br_ref,
                   xl_ref, xr_ref):
    h = _leaky(_mm_t(x_ref[...], W0_ref[...]) + b0_ref[...], 0.01)
    xl_ref[...] = _mm_t(h, Wl_ref[...]) + bl_ref[...]
    xr_ref[...] = _mm_t(h, Wr_ref[...]) + br_ref[...]


def _tc_proj2_body(lo_ref, hi_ref, bias_ref, Wl_ref, bl_ref, Wr_ref, br_ref,
                   xl_ref, xr_ref):
    h = _leaky(jnp.concatenate([lo_ref[...], hi_ref[...]], axis=1)
               + bias_ref[...], 0.01)
    xl_ref[...] = _mm_t(h, Wl_ref[...]) + bl_ref[...]
    xr_ref[...] = _mm_t(h, Wr_ref[...]) + br_ref[...]


def _tc_head_body(lo_ref, hi_ref, bias_ref, batch_ref,
                  fc1W_ref, fc1b_ref, fc2W_ref, fc2b_ref, out_ref):
    h3 = _leaky(jnp.concatenate([lo_ref[...], hi_ref[...]], axis=1)
                + bias_ref[...], 0.01)
    gid = lax.broadcasted_iota(jnp.int32, (G, 1), 0)
    onehot = (batch_ref[...] == gid).astype(jnp.float32)
    sums = lax.dot_general(onehot, h3, (((1,), (0,)), ((), ())),
                           preferred_element_type=jnp.float32)
    counts = jnp.sum(onehot, axis=1, keepdims=True)
    hg = sums / jnp.maximum(counts, 1.0)
    z1 = _leaky(_mm_t(hg, fc1W_ref[...]) + fc1b_ref[...], 0.01)
    out_ref[...] = _mm_t(z1, fc2W_ref[...]) + fc2b_ref[...]


_w_spec = pl.BlockSpec((D, D), lambda i: (0, 0))
_b_spec = pl.BlockSpec((1, D), lambda i: (0, 0))
_r_spec = pl.BlockSpec((TB, D), lambda i: (i, 0))

_tc_proj1 = pl.pallas_call(
    _tc_proj1_body,
    grid=(NP // TB,),
    in_specs=[_r_spec, _w_spec, _b_spec, _w_spec, _b_spec, _w_spec, _b_spec],
    out_specs=[_r_spec, _r_spec],
    out_shape=[jax.ShapeDtypeStruct((NP, D), jnp.float32)] * 2,
)

_h_spec = pl.BlockSpec((TB, DH), lambda i: (i, 0))

_tc_proj2 = pl.pallas_call(
    _tc_proj2_body,
    grid=(NP // TB,),
    in_specs=[_h_spec, _h_spec, _b_spec, _w_spec, _b_spec, _w_spec, _b_spec],
    out_specs=[_r_spec, _r_spec],
    out_shape=[jax.ShapeDtypeStruct((NP, D), jnp.float32)] * 2,
)

_tc_head = pl.pallas_call(
    _tc_head_body,
    out_shape=jax.ShapeDtypeStruct((G, D_OUT), jnp.float32),
)


def kernel(x, edge_index, batch, nfc_W, nfc_b,
           gc1_Wl, gc1_bl, gc1_Wr, gc1_br, gc1_att, gc1_bias,
           gc2_Wl, gc2_bl, gc2_Wr, gc2_br, gc2_att, gc2_bias,
           fc1_W, fc1_b, fc2_W, fc2_b):
    x_p = jnp.pad(x, ((0, NP - N), (0, 0)))
    src2d = edge_index[0].reshape(NW, CPW, K)
    dst2d = edge_index[1].reshape(NW, CPW, K)
    src3d = edge_index[0].reshape(NS, CPT, K)
    dst3d = edge_index[1].reshape(NS, CPT, K)
    batch2d = jnp.pad(batch, (0, NP - N), constant_values=G).reshape(1, NP)

    def b2(v):
        return v.reshape(1, -1)

    xl1, xr1 = _tc_proj1(x_p, nfc_W, b2(nfc_b), gc1_Wl, b2(gc1_bl),
                         gc1_Wr, b2(gc1_br))
    logits1, tmax1 = _sc_logits(xl1, xr1, src2d, dst2d, gc1_att)
    outp1 = _sc_aggregate(xl1.reshape(2 * NP, DH), src3d, dst3d,
                          logits1.reshape(NS, CPT, K), tmax1)

    xl2, xr2 = _tc_proj2(outp1[0], outp1[1],
                         b2(gc1_bias), gc2_Wl, b2(gc2_bl), gc2_Wr, b2(gc2_br))
    logits2, tmax2 = _sc_logits(xl2, xr2, src2d, dst2d, gc2_att)
    outp2 = _sc_aggregate(xl2.reshape(2 * NP, DH), src3d, dst3d,
                          logits2.reshape(NS, CPT, K), tmax2)

    return _tc_head(outp2[0], outp2[1],
                    b2(gc2_bias), batch2d, fc1_W, b2(fc1_b), fc2_W, b2(fc2_b))


# speedup vs baseline: 18.0705x; 1.0237x over previous
import functools

import jax
import jax.numpy as jnp
from jax import lax
from jax.experimental import pallas as pl
from jax.experimental.pallas import tpu as pltpu
from jax.experimental.pallas import tpu_sc as plsc

N = 10000
E = 320000
G = 16
D = 128
D_FC1 = 32
D_OUT = 10
NP = 10240
NC = 2
NS = 16
NW = NC * NS
EPW = E // NW
K = 80
CPW = EPW // K
TB = 512

_mesh = plsc.VectorSubcoreMesh(core_axis_name="c", subcore_axis_name="s")
_sc_params = pltpu.CompilerParams(needs_layout_passes=False, use_tc_tiling_on_sc=False)


def _leaky(v, slope):
    return jnp.maximum(v, v * slope)


@functools.partial(
    pl.kernel,
    out_type=[
        jax.ShapeDtypeStruct((NW, CPW, K), jnp.float32),
        jax.ShapeDtypeStruct((NW * 16,), jnp.float32),
    ],
    mesh=_mesh,
    compiler_params=_sc_params,
    scratch_types=[
        pltpu.VMEM((CPW, K), jnp.int32),
        pltpu.VMEM((CPW, K), jnp.int32),
        pltpu.VMEM((D,), jnp.float32),
        pltpu.VMEM((K, D), jnp.float32),
        pltpu.VMEM((K, D), jnp.float32),
        pltpu.VMEM((K, D), jnp.float32),
        pltpu.VMEM((K, D), jnp.float32),
        pltpu.VMEM((CPW, K), jnp.float32),
        pltpu.VMEM((16,), jnp.float32),
        pltpu.SemaphoreType.DMA,
    ],
)
def _sc_logits(xl_hbm, xr_hbm, src_hbm, dst_hbm, att_hbm,
               logits_hbm, tmax_hbm,
               src_v, dst_v, att_v, xl0_v, xl1_v, xr0_v, xr1_v,
               log_v, red_v, sem):
    wid = lax.axis_index("s") * NC + lax.axis_index("c")
    cp1 = pltpu.async_copy(src_hbm.at[wid], src_v, sem)
    cp2 = pltpu.async_copy(dst_hbm.at[wid], dst_v, sem)
    cp3 = pltpu.async_copy(att_hbm, att_v, sem)
    cp1.wait(); cp2.wait(); cp3.wait()

    lane = lax.iota(jnp.int32, 16)

    def _compute(c, xl_v, xr_v, rmax):
        @pl.loop(0, K // 16)
        def egrp(e16):
            lv = jnp.zeros((16,), jnp.float32)
            for l in range(16):
                e = e16 * 16 + l
                acc = jnp.zeros((16,), jnp.float32)
                for j in range(D // 16):
                    s = xl_v[e, pl.ds(j * 16, 16)] + xr_v[e, pl.ds(j * 16, 16)]
                    acc = acc + _leaky(s, 0.2) * att_v[pl.ds(j * 16, 16)]
                lv = jnp.where(lane == l, plsc.cumsum(acc)[15], lv)
            log_v[c, pl.ds(e16 * 16, 16)] = lv

        for q in range(K // 16):
            rmax = jnp.maximum(rmax, log_v[c, pl.ds(q * 16, 16)])
        return rmax

    def _gather(c, xl_v, xr_v):
        pltpu.async_copy(xl_hbm.at[src_v.at[c]], xl_v, sem)
        pltpu.async_copy(xr_hbm.at[dst_v.at[c]], xr_v, sem)

    def _drain(c, xl_v, xr_v):
        pltpu.make_async_copy(xl_hbm.at[src_v.at[c]], xl_v, sem).wait()
        pltpu.make_async_copy(xr_hbm.at[dst_v.at[c]], xr_v, sem).wait()

    _gather(0, xl0_v, xr0_v)

    @pl.loop(0, CPW // 2, init_carry=jnp.full((16,), -1e30, jnp.float32))
    def pair(p, rmax):
        c0 = p * 2
        _gather(c0 + 1, xl1_v, xr1_v)
        _drain(c0, xl0_v, xr0_v)
        rmax = _compute(c0, xl0_v, xr0_v, rmax)
        _gather(c0 + 2, xl0_v, xr0_v)
        _drain(c0 + 1, xl1_v, xr1_v)
        return _compute(c0 + 1, xl1_v, xr1_v, rmax)

    _drain(CPW - 1, xl0_v, xr0_v)
    red_v[...] = _compute(CPW - 1, xl0_v, xr0_v, pair)
    pltpu.sync_copy(red_v, tmax_hbm.at[pl.ds(wid * 16, 16)])
    pltpu.sync_copy(log_v, logits_hbm.at[wid])


CPT = NW * CPW // NS
DH = D // 2
NPB = NP // NS
WB = NPB // K


@functools.partial(
    pl.kernel,
    out_type=jax.ShapeDtypeStruct((NC, NP, DH), jnp.float32),
    mesh=_mesh,
    compiler_params=_sc_params,
    scratch_types=[
        pltpu.VMEM((CPT, K), jnp.int32),
        pltpu.VMEM((CPT, K), jnp.int32),
        pltpu.VMEM((CPT, K), jnp.float32),
        pltpu.VMEM((NW * 16,), jnp.float32),
        pltpu.VMEM((NPB,), jnp.float32),
        pltpu.VMEM((K, DH), jnp.float32),
        pltpu.VMEM((K, DH), jnp.float32),
        pltpu.VMEM((K, DH), jnp.float32),
        pltpu.VMEM((K, DH), jnp.float32),
        pltpu.VMEM_SHARED((NP, DH), jnp.float32),
        pltpu.VMEM_SHARED((NP,), jnp.float32),
        pltpu.SemaphoreType.DMA,
        pltpu.SemaphoreType.DMA,
        pltpu.SemaphoreType.DMA,
    ],
)
def _sc_aggregate(xlh_hbm, src_hbm, dst_hbm, logits_hbm, tmax_hbm,
                  outp_hbm,
                  src_v, dst_v, ex_v, tmax_v, den_v, xlr0_v, xlr1_v,
                  stage0_v, stage1_v, spout, spden, sem, sem2, sem3):
    cid = lax.axis_index("c")
    sid = lax.axis_index("s")
    r0 = sid * NPB
    cp1 = pltpu.async_copy(src_hbm.at[sid], src_v, sem)
    cp2 = pltpu.async_copy(dst_hbm.at[sid], dst_v, sem)
    cp3 = pltpu.async_copy(logits_hbm.at[sid], ex_v, sem)
    cp4 = pltpu.async_copy(tmax_hbm, tmax_v, sem)
    cp1.wait(); cp2.wait(); cp3.wait(); cp4.wait()

    m = tmax_v[pl.ds(0, 16)]
    for i in range(1, NW):
        m = jnp.maximum(m, tmax_v[pl.ds(i * 16, 16)])
    gmax = m[0]
    for l in range(1, 16):
        gmax = jnp.maximum(gmax, m[l])

    @pl.loop(0, CPT)
    def toex(c):
        for q in range(K // 16):
            lv = ex_v[c, pl.ds(q * 16, 16)]
            ex_v[c, pl.ds(q * 16, 16)] = jnp.exp(lv - gmax)

    @pl.loop(0, CPT)
    def fixsrc(c):
        for q in range(K // 16):
            v = src_v[c, pl.ds(q * 16, 16)]
            src_v[c, pl.ds(q * 16, 16)] = v * 2 + cid

    @pl.loop(0, K)
    def zer(e):
        for j in range(DH // 16):
            stage0_v[e, pl.ds(j * 16, 16)] = jnp.zeros((16,), jnp.float32)

    @pl.loop(0, NPB // 16)
    def zerd(i):
        den_v[pl.ds(i * 16, 16)] = jnp.zeros((16,), jnp.float32)

    for b in range(WB):
        pltpu.sync_copy(stage0_v, spout.at[pl.ds(r0 + b * K, K)])
    pltpu.sync_copy(den_v, spden.at[pl.ds(r0, NPB)])

    pltpu.async_copy(xlh_hbm.at[src_v.at[0]], xlr0_v, sem)

    plsc.subcore_barrier()

    def _drain_g(buf):
        pltpu.make_async_copy(xlh_hbm.at[src_v.at[0]], buf, sem).wait()

    def _drain_s(buf):
        pltpu.make_async_copy(xlh_hbm.at[src_v.at[0]], buf, sem2).wait()

    def _drain_e():
        pltpu.make_async_copy(ex_v.at[0], spden.at[dst_v.at[0]], sem3).wait()

    def _do_chunk(c, xlr_v, stage_v):
        @pl.loop(0, K // 16)
        def egrp(e16):
            av = ex_v[c, pl.ds(e16 * 16, 16)]
            for l in range(16):
                e = e16 * 16 + l
                a = av[l]
                for j in range(DH // 16):
                    stage_v[e, pl.ds(j * 16, 16)] = xlr_v[e, pl.ds(j * 16, 16)] * a

        pltpu.async_copy(ex_v.at[c], spden.at[dst_v.at[c]], sem3, add=True)

        pltpu.async_copy(stage_v, spout.at[dst_v.at[c]], sem2, add=True)

    @pl.loop(0, CPT // 2)
    def pair(p):
        c0 = p * 2
        pltpu.async_copy(xlh_hbm.at[src_v.at[c0 + 1]], xlr1_v, sem)
        _drain_g(xlr0_v)

        @pl.when(p > 0)
        def dr0():
            _drain_s(stage0_v)
            _drain_e()

        _do_chunk(c0, xlr0_v, stage0_v)

        @pl.when(c0 + 2 < CPT)
        def pre():
            pltpu.async_copy(xlh_hbm.at[src_v.at[c0 + 2]], xlr0_v, sem)

        _drain_g(xlr1_v)

        @pl.when(p > 0)
        def dr1():
            _drain_s(stage1_v)
            _drain_e()

        _do_chunk(c0 + 1, xlr1_v, stage1_v)

    _drain_s(stage0_v)
    _drain_s(stage1_v)
    _drain_e()
    _drain_e()
    plsc.subcore_barrier()

    pltpu.sync_copy(spden.at[pl.ds(r0, NPB)], den_v)

    @pl.loop(0, NPB // 16)
    def inv(i):
        v = den_v[pl.ds(i * 16, 16)]
        den_v[pl.ds(i * 16, 16)] = jnp.float32(1.0) / (v + jnp.float32(1e-16))

    for b in range(WB):
        stg = stage0_v if b % 2 == 0 else stage1_v
        if b >= 2:
            _drain_s(stg)
        pltpu.sync_copy(spout.at[pl.ds(r0 + b * K, K)], stg)
        for g in range(K // 16):
            iv = den_v[pl.ds(b * K + g * 16, 16)]
            for l in range(16):
                e = g * 16 + l
                a = iv[l]
                for j in range(DH // 16):
                    stg[e, pl.ds(j * 16, 16)] = stg[e, pl.ds(j * 16, 16)] * a
        pltpu.async_copy(stg, outp_hbm.at[cid, pl.ds(r0 + b * K, K)], sem2)

    _drain_s(stage0_v)
    _drain_s(stage1_v)


def _mm_t(a, w):
    return lax.dot_general(a, w, (((1,), (1,)), ((), ())),
                           preferred_element_type=jnp.float32)


def _tc_proj1_body(x_ref, W0_ref, b0_ref, Wl_ref, bl_ref, Wr_ref, br_ref,
                   xl_ref, xr_ref):
    h = _leaky(_mm_t(x_ref[...], W0_ref[...]) + b0_ref[...], 0.01)
    xl_ref[...] = _mm_t(h, Wl_ref[...]) + bl_ref[...]
    xr_ref[...] = _mm_t(h, Wr_ref[...]) + br_ref[...]


def _tc_proj2_body(lo_ref, hi_ref, bias_ref, Wl_ref, bl_ref, Wr_ref, br_ref,
                   xl_ref, xr_ref):
    h = _leaky(jnp.concatenate([lo_ref[...], hi_ref[...]], axis=1)
               + bias_ref[...], 0.01)
    xl_ref[...] = _mm_t(h, Wl_ref[...]) + bl_ref[...]
    xr_ref[...] = _mm_t(h, Wr_ref[...]) + br_ref[...]


def _tc_head_body(lo_ref, hi_ref, bias_ref, batch_ref,
                  fc1W_ref, fc1b_ref, fc2W_ref, fc2b_ref, out_ref):
    h3 = _leaky(jnp.concatenate([lo_ref[...], hi_ref[...]], axis=1)
                + bias_ref[...], 0.01)
    gid = lax.broadcasted_iota(jnp.int32, (G, 1), 0)
    onehot = (batch_ref[...] == gid).astype(jnp.float32)
    sums = lax.dot_general(onehot, h3, (((1,), (0,)), ((), ())),
                           preferred_element_type=jnp.float32)
    counts = jnp.sum(onehot, axis=1, keepdims=True)
    hg = sums / jnp.maximum(counts, 1.0)
    z1 = _leaky(_mm_t(hg, fc1W_ref[...]) + fc1b_ref[...], 0.01)
    out_ref[...] = _mm_t(z1, fc2W_ref[...]) + fc2b_ref[...]


_w_spec = pl.BlockSpec((D, D), lambda i: (0, 0))
_b_spec = pl.BlockSpec((1, D), lambda i: (0, 0))
_r_spec = pl.BlockSpec((TB, D), lambda i: (i, 0))

_tc_proj1 = pl.pallas_call(
    _tc_proj1_body,
    grid=(NP // TB,),
    in_specs=[_r_spec, _w_spec, _b_spec, _w_spec, _b_spec, _w_spec, _b_spec],
    out_specs=[_r_spec, _r_spec],
    out_shape=[jax.ShapeDtypeStruct((NP, D), jnp.float32)] * 2,
)

_h_spec = pl.BlockSpec((TB, DH), lambda i: (i, 0))

_tc_proj2 = pl.pallas_call(
    _tc_proj2_body,
    grid=(NP // TB,),
    in_specs=[_h_spec, _h_spec, _b_spec, _w_spec, _b_spec, _w_spec, _b_spec],
    out_specs=[_r_spec, _r_spec],
    out_shape=[jax.ShapeDtypeStruct((NP, D), jnp.float32)] * 2,
)

_tc_head = pl.pallas_call(
    _tc_head_body,
    out_shape=jax.ShapeDtypeStruct((G, D_OUT), jnp.float32),
)


def kernel(x, edge_index, batch, nfc_W, nfc_b,
           gc1_Wl, gc1_bl, gc1_Wr, gc1_br, gc1_att, gc1_bias,
           gc2_Wl, gc2_bl, gc2_Wr, gc2_br, gc2_att, gc2_bias,
           fc1_W, fc1_b, fc2_W, fc2_b):
    x_p = jnp.pad(x, ((0, NP - N), (0, 0)))
    src2d = edge_index[0].reshape(NW, CPW, K)
    dst2d = edge_index[1].reshape(NW, CPW, K)
    src3d = edge_index[0].reshape(NS, CPT, K)
    dst3d = edge_index[1].reshape(NS, CPT, K)
    batch2d = jnp.pad(batch, (0, NP - N), constant_values=G).reshape(1, NP)

    def b2(v):
        return v.reshape(1, -1)

    xl1, xr1 = _tc_proj1(x_p, nfc_W, b2(nfc_b), gc1_Wl, b2(gc1_bl),
                         gc1_Wr, b2(gc1_br))
    logits1, tmax1 = _sc_logits(xl1, xr1, src2d, dst2d, gc1_att)
    outp1 = _sc_aggregate(xl1.reshape(2 * NP, DH), src3d, dst3d,
                          logits1.reshape(NS, CPT, K), tmax1)

    xl2, xr2 = _tc_proj2(outp1[0], outp1[1],
                         b2(gc1_bias), gc2_Wl, b2(gc2_bl), gc2_Wr, b2(gc2_br))
    logits2, tmax2 = _sc_logits(xl2, xr2, src2d, dst2d, gc2_att)
    outp2 = _sc_aggregate(xl2.reshape(2 * NP, DH), src3d, dst3d,
                          logits2.reshape(NS, CPT, K), tmax2)

    return _tc_head(outp2[0], outp2[1],
                    b2(gc2_bias), batch2d, fc1_W, b2(fc1_b), fc2_W, b2(fc2_b))


# speedup vs baseline: 20.1495x; 1.1151x over previous
import functools

import jax
import jax.numpy as jnp
from jax import lax
from jax.experimental import pallas as pl
from jax.experimental.pallas import tpu as pltpu
from jax.experimental.pallas import tpu_sc as plsc

N = 10000
E = 320000
G = 16
D = 128
D_FC1 = 32
D_OUT = 10
NP = 10240
NC = 2
NS = 16
NW = NC * NS
EPW = E // NW
K = 80
CPW = EPW // K
TB = 512

_mesh = plsc.VectorSubcoreMesh(core_axis_name="c", subcore_axis_name="s")
_sc_params = pltpu.CompilerParams(needs_layout_passes=False, use_tc_tiling_on_sc=False)


def _leaky(v, slope):
    return jnp.maximum(v, v * slope)


@functools.partial(
    pl.kernel,
    out_type=[
        jax.ShapeDtypeStruct((NW, CPW, K), jnp.float32),
        jax.ShapeDtypeStruct((NW * 16,), jnp.float32),
    ],
    mesh=_mesh,
    compiler_params=_sc_params,
    scratch_types=[
        pltpu.VMEM((CPW, K), jnp.int32),
        pltpu.VMEM((CPW, K), jnp.int32),
        pltpu.VMEM((D,), jnp.float32),
        pltpu.VMEM((K, D), jnp.float32),
        pltpu.VMEM((K, D), jnp.float32),
        pltpu.VMEM((K, D), jnp.float32),
        pltpu.VMEM((K, D), jnp.float32),
        pltpu.VMEM((K, D), jnp.float32),
        pltpu.VMEM((K, D), jnp.float32),
        pltpu.VMEM((CPW, K), jnp.float32),
        pltpu.VMEM((16,), jnp.float32),
        pltpu.SemaphoreType.DMA,
    ],
)
def _sc_logits(xl_hbm, xr_hbm, src_hbm, dst_hbm, att_hbm,
               logits_hbm, tmax_hbm,
               src_v, dst_v, att_v, xl0_v, xl1_v, xl2_v, xr0_v, xr1_v, xr2_v,
               log_v, red_v, sem):
    wid = lax.axis_index("s") * NC + lax.axis_index("c")
    cp1 = pltpu.async_copy(src_hbm.at[wid], src_v, sem)
    cp2 = pltpu.async_copy(dst_hbm.at[wid], dst_v, sem)
    cp3 = pltpu.async_copy(att_hbm, att_v, sem)
    cp1.wait(); cp2.wait(); cp3.wait()

    lane = lax.iota(jnp.int32, 16)

    def _compute(c, xl_v, xr_v, rmax):
        @pl.loop(0, K // 16)
        def egrp(e16):
            lv = jnp.zeros((16,), jnp.float32)
            for l in range(16):
                e = e16 * 16 + l
                acc = jnp.zeros((16,), jnp.float32)
                for j in range(D // 16):
                    s = xl_v[e, pl.ds(j * 16, 16)] + xr_v[e, pl.ds(j * 16, 16)]
                    acc = acc + _leaky(s, 0.2) * att_v[pl.ds(j * 16, 16)]
                lv = jnp.where(lane == l, plsc.cumsum(acc)[15], lv)
            log_v[c, pl.ds(e16 * 16, 16)] = lv

        for q in range(K // 16):
            rmax = jnp.maximum(rmax, log_v[c, pl.ds(q * 16, 16)])
        return rmax

    def _gather(c, xl_v, xr_v):
        pltpu.async_copy(xl_hbm.at[src_v.at[c]], xl_v, sem)
        pltpu.async_copy(xr_hbm.at[dst_v.at[c]], xr_v, sem)

    def _drain(c, xl_v, xr_v):
        pltpu.make_async_copy(xl_hbm.at[src_v.at[c]], xl_v, sem).wait()
        pltpu.make_async_copy(xr_hbm.at[dst_v.at[c]], xr_v, sem).wait()

    _gather(0, xl0_v, xr0_v)
    _gather(1, xl1_v, xr1_v)

    @pl.loop(0, CPW // 3, init_carry=jnp.full((16,), -1e30, jnp.float32))
    def triple(p, rmax):
        c0 = p * 3
        _gather(c0 + 2, xl2_v, xr2_v)
        _drain(c0, xl0_v, xr0_v)
        rmax = _compute(c0, xl0_v, xr0_v, rmax)
        _gather(c0 + 3, xl0_v, xr0_v)
        _drain(c0 + 1, xl1_v, xr1_v)
        rmax = _compute(c0 + 1, xl1_v, xr1_v, rmax)
        _gather(c0 + 4, xl1_v, xr1_v)
        _drain(c0 + 2, xl2_v, xr2_v)
        return _compute(c0 + 2, xl2_v, xr2_v, rmax)

    _drain(CPW - 2, xl0_v, xr0_v)
    rmax2 = _compute(CPW - 2, xl0_v, xr0_v, triple)
    _drain(CPW - 1, xl1_v, xr1_v)
    red_v[...] = _compute(CPW - 1, xl1_v, xr1_v, rmax2)
    pltpu.sync_copy(red_v, tmax_hbm.at[pl.ds(wid * 16, 16)])
    pltpu.sync_copy(log_v, logits_hbm.at[wid])


CPT = NW * CPW // NS
DH = D // 2
NPB = NP // NS
WB = NPB // K


@functools.partial(
    pl.kernel,
    out_type=jax.ShapeDtypeStruct((NC, NP, DH), jnp.float32),
    mesh=_mesh,
    compiler_params=_sc_params,
    scratch_types=[
        pltpu.VMEM((CPT, K), jnp.int32),
        pltpu.VMEM((CPT, K), jnp.int32),
        pltpu.VMEM((CPT, K), jnp.float32),
        pltpu.VMEM((NW * 16,), jnp.float32),
        pltpu.VMEM((NPB,), jnp.float32),
        pltpu.VMEM((K, DH), jnp.float32),
        pltpu.VMEM((K, DH), jnp.float32),
        pltpu.VMEM((K, DH), jnp.float32),
        pltpu.VMEM((K, DH), jnp.float32),
        pltpu.VMEM_SHARED((NP, DH), jnp.float32),
        pltpu.VMEM_SHARED((NP,), jnp.float32),
        pltpu.SemaphoreType.DMA,
        pltpu.SemaphoreType.DMA,
        pltpu.SemaphoreType.DMA,
    ],
)
def _sc_aggregate(xlh_hbm, src_hbm, dst_hbm, logits_hbm, tmax_hbm,
                  outp_hbm,
                  src_v, dst_v, ex_v, tmax_v, den_v, xlr0_v, xlr1_v,
                  stage0_v, stage1_v, spout, spden, sem, sem2, sem3):
    cid = lax.axis_index("c")
    sid = lax.axis_index("s")
    r0 = sid * NPB
    cp1 = pltpu.async_copy(src_hbm.at[sid], src_v, sem)
    cp2 = pltpu.async_copy(dst_hbm.at[sid], dst_v, sem)
    cp3 = pltpu.async_copy(logits_hbm.at[sid], ex_v, sem)
    cp4 = pltpu.async_copy(tmax_hbm, tmax_v, sem)
    cp1.wait(); cp2.wait(); cp3.wait(); cp4.wait()

    m = tmax_v[pl.ds(0, 16)]
    for i in range(1, NW):
        m = jnp.maximum(m, tmax_v[pl.ds(i * 16, 16)])
    gmax = m[0]
    for l in range(1, 16):
        gmax = jnp.maximum(gmax, m[l])

    @pl.loop(0, CPT)
    def toex(c):
        for q in range(K // 16):
            lv = ex_v[c, pl.ds(q * 16, 16)]
            ex_v[c, pl.ds(q * 16, 16)] = jnp.exp(lv - gmax)

    @pl.loop(0, CPT)
    def fixsrc(c):
        for q in range(K // 16):
            v = src_v[c, pl.ds(q * 16, 16)]
            src_v[c, pl.ds(q * 16, 16)] = v * 2 + cid

    @pl.loop(0, K)
    def zer(e):
        for j in range(DH // 16):
            stage0_v[e, pl.ds(j * 16, 16)] = jnp.zeros((16,), jnp.float32)

    @pl.loop(0, NPB // 16)
    def zerd(i):
        den_v[pl.ds(i * 16, 16)] = jnp.zeros((16,), jnp.float32)

    for b in range(WB):
        pltpu.sync_copy(stage0_v, spout.at[pl.ds(r0 + b * K, K)])
    pltpu.sync_copy(den_v, spden.at[pl.ds(r0, NPB)])

    pltpu.async_copy(xlh_hbm.at[src_v.at[0]], xlr0_v, sem)

    plsc.subcore_barrier()

    def _drain_g(buf):
        pltpu.make_async_copy(xlh_hbm.at[src_v.at[0]], buf, sem).wait()

    def _drain_s(buf):
        pltpu.make_async_copy(xlh_hbm.at[src_v.at[0]], buf, sem2).wait()

    def _drain_e():
        pltpu.make_async_copy(ex_v.at[0], spden.at[dst_v.at[0]], sem3).wait()

    def _do_chunk(c, xlr_v, stage_v):
        @pl.loop(0, K // 16)
        def egrp(e16):
            av = ex_v[c, pl.ds(e16 * 16, 16)]
            for l in range(16):
                e = e16 * 16 + l
                a = av[l]
                for j in range(DH // 16):
                    stage_v[e, pl.ds(j * 16, 16)] = xlr_v[e, pl.ds(j * 16, 16)] * a

        pltpu.async_copy(ex_v.at[c], spden.at[dst_v.at[c]], sem3, add=True)

        pltpu.async_copy(stage_v, spout.at[dst_v.at[c]], sem2, add=True)

    @pl.loop(0, CPT // 2)
    def pair(p):
        c0 = p * 2
        pltpu.async_copy(xlh_hbm.at[src_v.at[c0 + 1]], xlr1_v, sem)
        _drain_g(xlr0_v)

        @pl.when(p > 0)
        def dr0():
            _drain_s(stage0_v)
            _drain_e()

        _do_chunk(c0, xlr0_v, stage0_v)

        @pl.when(c0 + 2 < CPT)
        def pre():
            pltpu.async_copy(xlh_hbm.at[src_v.at[c0 + 2]], xlr0_v, sem)

        _drain_g(xlr1_v)

        @pl.when(p > 0)
        def dr1():
            _drain_s(stage1_v)
            _drain_e()

        _do_chunk(c0 + 1, xlr1_v, stage1_v)

    _drain_s(stage0_v)
    _drain_s(stage1_v)
    _drain_e()
    _drain_e()
    plsc.subcore_barrier()

    pltpu.sync_copy(spden.at[pl.ds(r0, NPB)], den_v)

    @pl.loop(0, NPB // 16)
    def inv(i):
        v = den_v[pl.ds(i * 16, 16)]
        den_v[pl.ds(i * 16, 16)] = jnp.float32(1.0) / (v + jnp.float32(1e-16))

    for b in range(WB):
        stg = stage0_v if b % 2 == 0 else stage1_v
        if b >= 2:
            _drain_s(stg)
        pltpu.sync_copy(spout.at[pl.ds(r0 + b * K, K)], stg)
        for g in range(K // 16):
            iv = den_v[pl.ds(b * K + g * 16, 16)]
            for l in range(16):
                e = g * 16 + l
                a = iv[l]
                for j in range(DH // 16):
                    stg[e, pl.ds(j * 16, 16)] = stg[e, pl.ds(j * 16, 16)] * a
        pltpu.async_copy(stg, outp_hbm.at[cid, pl.ds(r0 + b * K, K)], sem2)

    _drain_s(stage0_v)
    _drain_s(stage1_v)


def _mm_t(a, w):
    return lax.dot_general(a, w, (((1,), (1,)), ((), ())),
                           preferred_element_type=jnp.float32)


def _tc_proj1_body(x_ref, W0_ref, b0_ref, Wl_ref, bl_ref, Wr_ref, br_ref,
                   xl_ref, xr_ref):
    h = _leaky(_mm_t(x_ref[...], W0_ref[...]) + b0_ref[...], 0.01)
    xl_ref[...] = _mm_t(h, Wl_ref[...]) + bl_ref[...]
    xr_ref[...] = _mm_t(h, Wr_ref[...]) + br_ref[...]


def _tc_proj2_body(lo_ref, hi_ref, bias_ref, Wl_ref, bl_ref, Wr_ref, br_ref,
                   xl_ref, xr_ref):
    h = _leaky(jnp.concatenate([lo_ref[...], hi_ref[...]], axis=1)
               + bias_ref[...], 0.01)
    xl_ref[...] = _mm_t(h, Wl_ref[...]) + bl_ref[...]
    xr_ref[...] = _mm_t(h, Wr_ref[...]) + br_ref[...]


def _tc_head_body(lo_ref, hi_ref, bias_ref, batch_ref,
                  fc1W_ref, fc1b_ref, fc2W_ref, fc2b_ref, out_ref):
    h3 = _leaky(jnp.concatenate([lo_ref[...], hi_ref[...]], axis=1)
                + bias_ref[...], 0.01)
    gid = lax.broadcasted_iota(jnp.int32, (G, 1), 0)
    onehot = (batch_ref[...] == gid).astype(jnp.float32)
    sums = lax.dot_general(onehot, h3, (((1,), (0,)), ((), ())),
                           preferred_element_type=jnp.float32)
    counts = jnp.sum(onehot, axis=1, keepdims=True)
    hg = sums / jnp.maximum(counts, 1.0)
    z1 = _leaky(_mm_t(hg, fc1W_ref[...]) + fc1b_ref[...], 0.01)
    out_ref[...] = _mm_t(z1, fc2W_ref[...]) + fc2b_ref[...]


_w_spec = pl.BlockSpec((D, D), lambda i: (0, 0))
_b_spec = pl.BlockSpec((1, D), lambda i: (0, 0))
_r_spec = pl.BlockSpec((TB, D), lambda i: (i, 0))

_tc_proj1 = pl.pallas_call(
    _tc_proj1_body,
    grid=(NP // TB,),
    in_specs=[_r_spec, _w_spec, _b_spec, _w_spec, _b_spec, _w_spec, _b_spec],
    out_specs=[_r_spec, _r_spec],
    out_shape=[jax.ShapeDtypeStruct((NP, D), jnp.float32)] * 2,
)

_h_spec = pl.BlockSpec((TB, DH), lambda i: (i, 0))

_tc_proj2 = pl.pallas_call(
    _tc_proj2_body,
    grid=(NP // TB,),
    in_specs=[_h_spec, _h_spec, _b_spec, _w_spec, _b_spec, _w_spec, _b_spec],
    out_specs=[_r_spec, _r_spec],
    out_shape=[jax.ShapeDtypeStruct((NP, D), jnp.float32)] * 2,
)

_tc_head = pl.pallas_call(
    _tc_head_body,
    out_shape=jax.ShapeDtypeStruct((G, D_OUT), jnp.float32),
)


def kernel(x, edge_index, batch, nfc_W, nfc_b,
           gc1_Wl, gc1_bl, gc1_Wr, gc1_br, gc1_att, gc1_bias,
           gc2_Wl, gc2_bl, gc2_Wr, gc2_br, gc2_att, gc2_bias,
           fc1_W, fc1_b, fc2_W, fc2_b):
    x_p = jnp.pad(x, ((0, NP - N), (0, 0)))
    src2d = edge_index[0].reshape(NW, CPW, K)
    dst2d = edge_index[1].reshape(NW, CPW, K)
    src3d = edge_index[0].reshape(NS, CPT, K)
    dst3d = edge_index[1].reshape(NS, CPT, K)
    batch2d = jnp.pad(batch, (0, NP - N), constant_values=G).reshape(1, NP)

    def b2(v):
        return v.reshape(1, -1)

    xl1, xr1 = _tc_proj1(x_p, nfc_W, b2(nfc_b), gc1_Wl, b2(gc1_bl),
                         gc1_Wr, b2(gc1_br))
    logits1, tmax1 = _sc_logits(xl1, xr1, src2d, dst2d, gc1_att)
    outp1 = _sc_aggregate(xl1.reshape(2 * NP, DH), src3d, dst3d,
                          logits1.reshape(NS, CPT, K), tmax1)

    xl2, xr2 = _tc_proj2(outp1[0], outp1[1],
                         b2(gc1_bias), gc2_Wl, b2(gc2_bl), gc2_Wr, b2(gc2_br))
    logits2, tmax2 = _sc_logits(xl2, xr2, src2d, dst2d, gc2_att)
    outp2 = _sc_aggregate(xl2.reshape(2 * NP, DH), src3d, dst3d,
                          logits2.reshape(NS, CPT, K), tmax2)

    return _tc_head(outp2[0], outp2[1],
                    b2(gc2_bias), batch2d, fc1_W, b2(fc1_b), fc2_W, b2(fc2_b))


# speedup vs baseline: 22.2333x; 1.1034x over previous
import functools

import jax
import jax.numpy as jnp
from jax import lax
from jax.experimental import pallas as pl
from jax.experimental.pallas import tpu as pltpu
from jax.experimental.pallas import tpu_sc as plsc

N = 10000
E = 320000
G = 16
D = 128
D_FC1 = 32
D_OUT = 10
NP = 10240
NC = 2
NS = 16
NW = NC * NS
EPW = E // NW
K = 80
CPW = EPW // K
TB = 512

_mesh = plsc.VectorSubcoreMesh(core_axis_name="c", subcore_axis_name="s")
_sc_params = pltpu.CompilerParams(needs_layout_passes=False, use_tc_tiling_on_sc=False)


def _leaky(v, slope):
    return jnp.maximum(v, v * slope)


@functools.partial(
    pl.kernel,
    out_type=[
        jax.ShapeDtypeStruct((NW, CPW, K), jnp.float32),
        jax.ShapeDtypeStruct((NW * 16,), jnp.float32),
    ],
    mesh=_mesh,
    compiler_params=_sc_params,
    scratch_types=[
        pltpu.VMEM((CPW, K), jnp.int32),
        pltpu.VMEM((CPW, K), jnp.int32),
        pltpu.VMEM((D,), jnp.float32),
        pltpu.VMEM((K, D), jnp.float32),
        pltpu.VMEM((K, D), jnp.float32),
        pltpu.VMEM((K, D), jnp.float32),
        pltpu.VMEM((K, D), jnp.float32),
        pltpu.VMEM((K, D), jnp.float32),
        pltpu.VMEM((K, D), jnp.float32),
        pltpu.VMEM((CPW, K), jnp.float32),
        pltpu.VMEM((16,), jnp.float32),
        pltpu.SemaphoreType.DMA,
    ],
)
def _sc_logits(xl_hbm, xr_hbm, src_hbm, dst_hbm, att_hbm,
               logits_hbm, tmax_hbm,
               src_v, dst_v, att_v, xl0_v, xl1_v, xl2_v, xr0_v, xr1_v, xr2_v,
               log_v, red_v, sem):
    wid = lax.axis_index("s") * NC + lax.axis_index("c")
    cp1 = pltpu.async_copy(src_hbm.at[wid], src_v, sem)
    cp2 = pltpu.async_copy(dst_hbm.at[wid], dst_v, sem)
    cp3 = pltpu.async_copy(att_hbm, att_v, sem)
    cp1.wait(); cp2.wait(); cp3.wait()

    lane = lax.iota(jnp.int32, 16)

    def _compute(c, xl_v, xr_v, rmax):
        @pl.loop(0, K // 16)
        def egrp(e16):
            lv = jnp.zeros((16,), jnp.float32)
            for l in range(16):
                e = e16 * 16 + l
                acc = jnp.zeros((16,), jnp.float32)
                for j in range(D // 16):
                    s = xl_v[e, pl.ds(j * 16, 16)] + xr_v[e, pl.ds(j * 16, 16)]
                    acc = acc + _leaky(s, 0.2) * att_v[pl.ds(j * 16, 16)]
                lv = jnp.where(lane == l, plsc.cumsum(acc)[15], lv)
            log_v[c, pl.ds(e16 * 16, 16)] = lv

        for q in range(K // 16):
            rmax = jnp.maximum(rmax, log_v[c, pl.ds(q * 16, 16)])
        return rmax

    def _gather(c, xl_v, xr_v):
        pltpu.async_copy(xl_hbm.at[src_v.at[c]], xl_v, sem)
        pltpu.async_copy(xr_hbm.at[dst_v.at[c]], xr_v, sem)

    def _drain(c, xl_v, xr_v):
        pltpu.make_async_copy(xl_hbm.at[src_v.at[c]], xl_v, sem).wait()
        pltpu.make_async_copy(xr_hbm.at[dst_v.at[c]], xr_v, sem).wait()

    _gather(0, xl0_v, xr0_v)
    _gather(1, xl1_v, xr1_v)

    @pl.loop(0, CPW // 3, init_carry=jnp.full((16,), -1e30, jnp.float32))
    def triple(p, rmax):
        c0 = p * 3
        _gather(c0 + 2, xl2_v, xr2_v)
        _drain(c0, xl0_v, xr0_v)
        rmax = _compute(c0, xl0_v, xr0_v, rmax)
        _gather(c0 + 3, xl0_v, xr0_v)
        _drain(c0 + 1, xl1_v, xr1_v)
        rmax = _compute(c0 + 1, xl1_v, xr1_v, rmax)
        _gather(c0 + 4, xl1_v, xr1_v)
        _drain(c0 + 2, xl2_v, xr2_v)
        return _compute(c0 + 2, xl2_v, xr2_v, rmax)

    _drain(CPW - 2, xl0_v, xr0_v)
    rmax2 = _compute(CPW - 2, xl0_v, xr0_v, triple)
    _drain(CPW - 1, xl1_v, xr1_v)
    red_v[...] = _compute(CPW - 1, xl1_v, xr1_v, rmax2)
    pltpu.sync_copy(red_v, tmax_hbm.at[pl.ds(wid * 16, 16)])
    pltpu.sync_copy(log_v, logits_hbm.at[wid])


CPT = NW * CPW // NS
DH = D // 2
NPB = NP // NS
WB = NPB // K


@functools.partial(
    pl.kernel,
    out_type=jax.ShapeDtypeStruct((NC, NP, DH), jnp.float32),
    mesh=_mesh,
    compiler_params=_sc_params,
    scratch_types=[
        pltpu.VMEM((CPT, K), jnp.int32),
        pltpu.VMEM((CPT, K), jnp.int32),
        pltpu.VMEM((CPT, K), jnp.float32),
        pltpu.VMEM((NW * 16,), jnp.float32),
        pltpu.VMEM((NPB,), jnp.float32),
        pltpu.VMEM((K, DH), jnp.float32),
        pltpu.VMEM((K, DH), jnp.float32),
        pltpu.VMEM((K, DH), jnp.float32),
        pltpu.VMEM((K, DH), jnp.float32),
        pltpu.VMEM((K, DH), jnp.float32),
        pltpu.VMEM_SHARED((NP, DH), jnp.float32),
        pltpu.VMEM_SHARED((NP,), jnp.float32),
        pltpu.SemaphoreType.DMA,
        pltpu.SemaphoreType.DMA,
        pltpu.SemaphoreType.DMA,
    ],
)
def _sc_aggregate(xlh_hbm, src_hbm, dst_hbm, logits_hbm, tmax_hbm,
                  outp_hbm,
                  src_v, dst_v, ex_v, tmax_v, den_v, xlr0_v, xlr1_v, xlr2_v,
                  stage0_v, stage1_v, spout, spden, sem, sem2, sem3):
    cid = lax.axis_index("c")
    sid = lax.axis_index("s")
    r0 = sid * NPB
    cp1 = pltpu.async_copy(src_hbm.at[sid], src_v, sem)
    cp2 = pltpu.async_copy(dst_hbm.at[sid], dst_v, sem)
    cp3 = pltpu.async_copy(logits_hbm.at[sid], ex_v, sem)
    cp4 = pltpu.async_copy(tmax_hbm, tmax_v, sem)
    cp1.wait(); cp2.wait(); cp3.wait(); cp4.wait()

    m = tmax_v[pl.ds(0, 16)]
    for i in range(1, NW):
        m = jnp.maximum(m, tmax_v[pl.ds(i * 16, 16)])
    gmax = m[0]
    for l in range(1, 16):
        gmax = jnp.maximum(gmax, m[l])

    @pl.loop(0, CPT)
    def toex(c):
        for q in range(K // 16):
            lv = ex_v[c, pl.ds(q * 16, 16)]
            ex_v[c, pl.ds(q * 16, 16)] = jnp.exp(lv - gmax)

    @pl.loop(0, CPT)
    def fixsrc(c):
        for q in range(K // 16):
            v = src_v[c, pl.ds(q * 16, 16)]
            src_v[c, pl.ds(q * 16, 16)] = v * 2 + cid

    @pl.loop(0, K)
    def zer(e):
        for j in range(DH // 16):
            stage0_v[e, pl.ds(j * 16, 16)] = jnp.zeros((16,), jnp.float32)

    @pl.loop(0, NPB // 16)
    def zerd(i):
        den_v[pl.ds(i * 16, 16)] = jnp.zeros((16,), jnp.float32)

    for b in range(WB):
        pltpu.sync_copy(stage0_v, spout.at[pl.ds(r0 + b * K, K)])
    pltpu.sync_copy(den_v, spden.at[pl.ds(r0, NPB)])

    pltpu.async_copy(xlh_hbm.at[src_v.at[0]], xlr0_v, sem)
    pltpu.async_copy(xlh_hbm.at[src_v.at[1]], xlr1_v, sem)

    plsc.subcore_barrier()

    def _drain_g(buf):
        pltpu.make_async_copy(xlh_hbm.at[src_v.at[0]], buf, sem).wait()

    def _drain_s(buf):
        pltpu.make_async_copy(xlh_hbm.at[src_v.at[0]], buf, sem2).wait()

    def _drain_e():
        pltpu.make_async_copy(ex_v.at[0], spden.at[dst_v.at[0]], sem3).wait()

    def _do_chunk(c, xlr_v, stage_v):
        @pl.loop(0, K // 16)
        def egrp(e16):
            av = ex_v[c, pl.ds(e16 * 16, 16)]
            for l in range(16):
                e = e16 * 16 + l
                a = av[l]
                for j in range(DH // 16):
                    stage_v[e, pl.ds(j * 16, 16)] = xlr_v[e, pl.ds(j * 16, 16)] * a

        pltpu.async_copy(ex_v.at[c], spden.at[dst_v.at[c]], sem3, add=True)

        pltpu.async_copy(stage_v, spout.at[dst_v.at[c]], sem2, add=True)

    xlrs = [xlr0_v, xlr1_v, xlr2_v]
    stages = [stage0_v, stage1_v]

    @pl.loop(0, CPT // 6)
    def six(p):
        c0 = p * 6
        for i in range(6):
            pltpu.async_copy(xlh_hbm.at[src_v.at[c0 + i + 2]],
                             xlrs[(i + 2) % 3], sem)
            _drain_g(xlrs[i % 3])
            if i < 2:
                @pl.when(p > 0)
                def dr():
                    _drain_s(stages[i % 2])
                    _drain_e()
            else:
                _drain_s(stages[i % 2])
                _drain_e()
            _do_chunk(c0 + i, xlrs[i % 3], stages[i % 2])

    e0 = CPT - 4
    pltpu.async_copy(xlh_hbm.at[src_v.at[e0 + 2]], xlrs[2], sem)
    _drain_g(xlrs[0])
    _drain_s(stages[0]); _drain_e()
    _do_chunk(e0, xlrs[0], stages[0])
    pltpu.async_copy(xlh_hbm.at[src_v.at[e0 + 3]], xlrs[0], sem)
    _drain_g(xlrs[1])
    _drain_s(stages[1]); _drain_e()
    _do_chunk(e0 + 1, xlrs[1], stages[1])
    _drain_g(xlrs[2])
    _drain_s(stages[0]); _drain_e()
    _do_chunk(e0 + 2, xlrs[2], stages[0])
    _drain_g(xlrs[0])
    _drain_s(stages[1]); _drain_e()
    _do_chunk(e0 + 3, xlrs[0], stages[1])
    _drain_s(stages[0]); _drain_e()
    _drain_s(stages[1]); _drain_e()
    plsc.subcore_barrier()

    pltpu.sync_copy(spden.at[pl.ds(r0, NPB)], den_v)

    @pl.loop(0, NPB // 16)
    def inv(i):
        v = den_v[pl.ds(i * 16, 16)]
        den_v[pl.ds(i * 16, 16)] = jnp.float32(1.0) / (v + jnp.float32(1e-16))

    for b in range(WB):
        stg = stage0_v if b % 2 == 0 else stage1_v
        if b >= 2:
            _drain_s(stg)
        pltpu.sync_copy(spout.at[pl.ds(r0 + b * K, K)], stg)
        for g in range(K // 16):
            iv = den_v[pl.ds(b * K + g * 16, 16)]
            for l in range(16):
                e = g * 16 + l
                a = iv[l]
                for j in range(DH // 16):
                    stg[e, pl.ds(j * 16, 16)] = stg[e, pl.ds(j * 16, 16)] * a
        pltpu.async_copy(stg, outp_hbm.at[cid, pl.ds(r0 + b * K, K)], sem2)

    _drain_s(stage0_v)
    _drain_s(stage1_v)


def _mm_t(a, w):
    return lax.dot_general(a, w, (((1,), (1,)), ((), ())),
                           preferred_element_type=jnp.float32)


def _tc_proj1_body(x_ref, W0_ref, b0_ref, Wl_ref, bl_ref, Wr_ref, br_ref,
                   xl_ref, xr_ref):
    h = _leaky(_mm_t(x_ref[...], W0_ref[...]) + b0_ref[...], 0.01)
    xl_ref[...] = _mm_t(h, Wl_ref[...]) + bl_ref[...]
    xr_ref[...] = _mm_t(h, Wr_ref[...]) + br_ref[...]


def _tc_proj2_body(lo_ref, hi_ref, bias_ref, Wl_ref, bl_ref, Wr_ref, br_ref,
                   xl_ref, xr_ref):
    h = _leaky(jnp.concatenate([lo_ref[...], hi_ref[...]], axis=1)
               + bias_ref[...], 0.01)
    xl_ref[...] = _mm_t(h, Wl_ref[...]) + bl_ref[...]
    xr_ref[...] = _mm_t(h, Wr_ref[...]) + br_ref[...]


def _tc_head_body(lo_ref, hi_ref, bias_ref, batch_ref,
                  fc1W_ref, fc1b_ref, fc2W_ref, fc2b_ref, out_ref):
    h3 = _leaky(jnp.concatenate([lo_ref[...], hi_ref[...]], axis=1)
                + bias_ref[...], 0.01)
    gid = lax.broadcasted_iota(jnp.int32, (G, 1), 0)
    onehot = (batch_ref[...] == gid).astype(jnp.float32)
    sums = lax.dot_general(onehot, h3, (((1,), (0,)), ((), ())),
                           preferred_element_type=jnp.float32)
    counts = jnp.sum(onehot, axis=1, keepdims=True)
    hg = sums / jnp.maximum(counts, 1.0)
    z1 = _leaky(_mm_t(hg, fc1W_ref[...]) + fc1b_ref[...], 0.01)
    out_ref[...] = _mm_t(z1, fc2W_ref[...]) + fc2b_ref[...]


_w_spec = pl.BlockSpec((D, D), lambda i: (0, 0))
_b_spec = pl.BlockSpec((1, D), lambda i: (0, 0))
_r_spec = pl.BlockSpec((TB, D), lambda i: (i, 0))

_tc_proj1 = pl.pallas_call(
    _tc_proj1_body,
    grid=(NP // TB,),
    in_specs=[_r_spec, _w_spec, _b_spec, _w_spec, _b_spec, _w_spec, _b_spec],
    out_specs=[_r_spec, _r_spec],
    out_shape=[jax.ShapeDtypeStruct((NP, D), jnp.float32)] * 2,
)

_h_spec = pl.BlockSpec((TB, DH), lambda i: (i, 0))

_tc_proj2 = pl.pallas_call(
    _tc_proj2_body,
    grid=(NP // TB,),
    in_specs=[_h_spec, _h_spec, _b_spec, _w_spec, _b_spec, _w_spec, _b_spec],
    out_specs=[_r_spec, _r_spec],
    out_shape=[jax.ShapeDtypeStruct((NP, D), jnp.float32)] * 2,
)

_tc_head = pl.pallas_call(
    _tc_head_body,
    out_shape=jax.ShapeDtypeStruct((G, D_OUT), jnp.float32),
)


def kernel(x, edge_index, batch, nfc_W, nfc_b,
           gc1_Wl, gc1_bl, gc1_Wr, gc1_br, gc1_att, gc1_bias,
           gc2_Wl, gc2_bl, gc2_Wr, gc2_br, gc2_att, gc2_bias,
           fc1_W, fc1_b, fc2_W, fc2_b):
    x_p = jnp.pad(x, ((0, NP - N), (0, 0)))
    src2d = edge_index[0].reshape(NW, CPW, K)
    dst2d = edge_index[1].reshape(NW, CPW, K)
    src3d = edge_index[0].reshape(NS, CPT, K)
    dst3d = edge_index[1].reshape(NS, CPT, K)
    batch2d = jnp.pad(batch, (0, NP - N), constant_values=G).reshape(1, NP)

    def b2(v):
        return v.reshape(1, -1)

    xl1, xr1 = _tc_proj1(x_p, nfc_W, b2(nfc_b), gc1_Wl, b2(gc1_bl),
                         gc1_Wr, b2(gc1_br))
    logits1, tmax1 = _sc_logits(xl1, xr1, src2d, dst2d, gc1_att)
    outp1 = _sc_aggregate(xl1.reshape(2 * NP, DH), src3d, dst3d,
                          logits1.reshape(NS, CPT, K), tmax1)

    xl2, xr2 = _tc_proj2(outp1[0], outp1[1],
                         b2(gc1_bias), gc2_Wl, b2(gc2_bl), gc2_Wr, b2(gc2_br))
    logits2, tmax2 = _sc_logits(xl2, xr2, src2d, dst2d, gc2_att)
    outp2 = _sc_aggregate(xl2.reshape(2 * NP, DH), src3d, dst3d,
                          logits2.reshape(NS, CPT, K), tmax2)

    return _tc_head(outp2[0], outp2[1],
                    b2(gc2_bias), batch2d, fc1_W, b2(fc1_b), fc2_W, b2(fc2_b))
